# Initial kernel scaffold; baseline (speedup 1.0000x reference)
import jax
import jax.numpy as jnp
from jax.experimental import pallas as pl


def kernel(x, edge_index, batch, full_edge_index, W_gcn, b_gcn, W_gcnx, b_gcnx, W_gcny, b_gcny, W_g1, b_g1, W_g2, b_g2, W_g3, b_g3, g0, be0, g1, be1, g2, be2, g3, be3, Wc1, bc1, Wc2, bc2):
    raise NotImplementedError("write your pallas kernel here")


# speedup vs baseline: 6.7821x; 6.7821x over previous
import functools

import jax
import jax.numpy as jnp
from jax import lax
from jax.experimental import pallas as pl
from jax.experimental.pallas import tpu as pltpu
from jax.experimental.pallas import tpu_sc as plsc

N = 10000
E = 320000
B = 1000
EF = 16000

NROWS = 10240
BROWS = 1024
DEGR = NROWS + 2 * BROWS
CHUNK = 128
NW = 32
EPS = 1e-5

E_BIG = 323584
E_POOL = 12288
E_SUP = 16384
E_DEG = 348160

@functools.lru_cache(maxsize=None)
def _mesh():
    return plsc.VectorSubcoreMesh(core_axis_name="c", subcore_axis_name="s")


def _fill_rows(ref, nrows, ncols, value):
    vec = jnp.full((16,), value, dtype=ref.dtype)

    def body(i, _):
        for j in range(ncols // 16):
            ref[i, pl.ds(j * 16, 16)] = vec
        return 0

    lax.fori_loop(0, nrows, body, 0)


@functools.lru_cache(maxsize=None)
def _make_count_scatter(n_edges, acc_rows):
    cpw = n_edges // (NW * CHUNK)
    rpt = acc_rows // 16
    nzc = rpt // CHUNK

    @functools.partial(
        pl.kernel,
        mesh=_mesh(),
        compiler_params=pltpu.CompilerParams(use_tc_tiling_on_sc=False),
        out_type=jax.ShapeDtypeStruct((2, acc_rows, 16), jnp.float32),
        scratch_types=[
            pltpu.VMEM_SHARED((acc_rows, 16), jnp.float32),
            pltpu.VMEM((CHUNK, 16), jnp.float32),
            pltpu.VMEM((CHUNK,), jnp.int32),
        ],
    )
    def count_kernel(idx_hbm, out_hbm, acc, ones_v, ibuf):
        c = lax.axis_index("c")
        s = lax.axis_index("s")
        _fill_rows(ones_v, CHUNK, 16, 0.0)

        def zero_body(k, _):
            pltpu.sync_copy(ones_v, acc.at[pl.ds(s * rpt + k * CHUNK, CHUNK)])
            return 0

        lax.fori_loop(0, nzc, zero_body, 0)
        plsc.subcore_barrier()
        _fill_rows(ones_v, CHUNK, 16, 1.0)
        wid = s * 2 + c
        base = wid * cpw * CHUNK

        def body(k, _):
            pltpu.sync_copy(idx_hbm.at[pl.ds(base + k * CHUNK, CHUNK)], ibuf)
            pltpu.sync_copy(ones_v, acc.at[ibuf], add=True)
            return 0

        lax.fori_loop(0, cpw, body, 0)
        plsc.subcore_barrier()

        def wb_body(k, _):
            r0 = s * rpt + k * CHUNK
            pltpu.sync_copy(acc.at[pl.ds(r0, CHUNK)], ones_v)
            pltpu.sync_copy(ones_v, out_hbm.at[c, pl.ds(r0, CHUNK)])
            return 0

        lax.fori_loop(0, nzc, wb_body, 0)

    return count_kernel


@functools.lru_cache(maxsize=None)
def _make_prop_scatter(n_edges, acc_rows, feat):
    cpw = n_edges // (NW * CHUNK)
    rpt = acc_rows // 16
    zc = min(CHUNK, rpt)
    nzc = rpt // zc

    @functools.partial(
        pl.kernel,
        mesh=_mesh(),
        compiler_params=pltpu.CompilerParams(use_tc_tiling_on_sc=False),
        out_type=jax.ShapeDtypeStruct((2, acc_rows, feat), jnp.float32),
        scratch_types=[
            pltpu.VMEM_SHARED((acc_rows, feat), jnp.float32),
            pltpu.VMEM((CHUNK, feat), jnp.float32),
            pltpu.VMEM((CHUNK,), jnp.int32),
            pltpu.VMEM((CHUNK,), jnp.int32),
            pltpu.SemaphoreType.DMA,
        ],
    )
    def prop_kernel(table_hbm, sidx_hbm, didx_hbm, out_hbm,
                    acc, rows_v, sbuf, dbuf, sem):
        c = lax.axis_index("c")
        s = lax.axis_index("s")
        _fill_rows(rows_v, zc, feat, 0.0)

        def zero_body(k, _):
            pltpu.sync_copy(rows_v.at[pl.ds(0, zc)],
                            acc.at[pl.ds(s * rpt + k * zc, zc)])
            return 0

        lax.fori_loop(0, nzc, zero_body, 0)
        plsc.subcore_barrier()
        wid = s * 2 + c
        base = wid * cpw * CHUNK

        def body(k, _):
            off = base + k * CHUNK
            pltpu.sync_copy(sidx_hbm.at[pl.ds(off, CHUNK)], sbuf)
            pltpu.sync_copy(didx_hbm.at[pl.ds(off, CHUNK)], dbuf)
            pltpu.async_copy(table_hbm.at[sbuf], rows_v, sem).wait()
            pltpu.sync_copy(rows_v, acc.at[dbuf], add=True)
            return 0

        lax.fori_loop(0, cpw, body, 0)
        plsc.subcore_barrier()

        def wb_body(k, _):
            r0 = s * rpt + k * zc
            pltpu.sync_copy(acc.at[pl.ds(r0, zc)], rows_v.at[pl.ds(0, zc)])
            pltpu.sync_copy(rows_v.at[pl.ds(0, zc)], out_hbm.at[c, pl.ds(r0, zc)])
            return 0

        lax.fori_loop(0, nzc, wb_body, 0)

    return prop_kernel


def _prep_scalars(cnt):
    def body(cnt_ref, dinv_ref, cinv_ref, dinvf_ref, deginvf_ref):
        col = cnt_ref[0, :, 0:1] + cnt_ref[1, :, 0:1]
        rb = lax.broadcasted_iota(jnp.int32, (BROWS, 1), 0)
        rn = lax.broadcasted_iota(jnp.int32, (NROWS, 1), 0)
        deg = col[0:NROWS] + 1.0
        dinv_ref[...] = jnp.where(rn < N, lax.rsqrt(deg), 0.0)
        cb = col[NROWS:NROWS + BROWS]
        cinv_ref[...] = jnp.where(rb < B, 1.0 / jnp.maximum(cb, 1.0), 0.0)
        degf = col[NROWS + BROWS:DEGR] + 1.0
        dinvf_ref[...] = jnp.where(rb < B, lax.rsqrt(degf), 0.0)
        deginvf_ref[...] = jnp.where(rb < B, 1.0 / degf, 0.0)

    one = jax.ShapeDtypeStruct((BROWS, 1), jnp.float32)
    return pl.pallas_call(
        body,
        out_shape=[jax.ShapeDtypeStruct((NROWS, 1), jnp.float32), one, one, one],
    )(cnt)


def _row_scale(xarr, dinv):
    rows, feat = xarr.shape
    blk = 512
    grid = rows // blk

    def body(x_ref, d_ref, o_ref):
        o_ref[...] = x_ref[...] * d_ref[...]

    return pl.pallas_call(
        body,
        grid=(grid,),
        in_specs=[
            pl.BlockSpec((blk, feat), lambda i: (i, 0)),
            pl.BlockSpec((blk, 1), lambda i: (i, 0)),
        ],
        out_specs=pl.BlockSpec((blk, feat), lambda i: (i, 0)),
        out_shape=jax.ShapeDtypeStruct((rows, feat), jnp.float32),
    )(xarr, dinv)


def _layer_big(v2, u, dinv, W, b):
    rows, cin = u.shape
    cout = W.shape[1]
    blk = 512
    grid = rows // blk

    def body(v_ref, u_ref, d_ref, w_ref, b_ref, h_ref, un_ref):
        p = (v_ref[0] + v_ref[1] + u_ref[...]) * d_ref[...]
        h = jnp.maximum(
            jnp.dot(p, w_ref[...], preferred_element_type=jnp.float32)
            + b_ref[...], 0.0)
        h_ref[...] = h
        un_ref[...] = h * d_ref[...]

    out = jax.ShapeDtypeStruct((rows, cout), jnp.float32)
    return pl.pallas_call(
        body,
        grid=(grid,),
        in_specs=[
            pl.BlockSpec((2, blk, cin), lambda i: (0, i, 0)),
            pl.BlockSpec((blk, cin), lambda i: (i, 0)),
            pl.BlockSpec((blk, 1), lambda i: (i, 0)),
            pl.BlockSpec((cin, cout), lambda i: (0, 0)),
            pl.BlockSpec((1, cout), lambda i: (0, 0)),
        ],
        out_specs=[
            pl.BlockSpec((blk, cout), lambda i: (i, 0)),
            pl.BlockSpec((blk, cout), lambda i: (i, 0)),
        ],
        out_shape=[out, out],
    )(v2, u, dinv, W, b)


def _pool_finish(P, cinv, g, be):
    feat = P.shape[2]

    def body(p_ref, c_ref, g_ref, be_ref, o_ref):
        pooled = (p_ref[0] + p_ref[1]) * c_ref[...]
        m = jnp.sum(pooled, axis=0, keepdims=True) / B
        var = jnp.sum(pooled * pooled, axis=0, keepdims=True) / B - m * m
        xn = (pooled - m) * lax.rsqrt(var + EPS) * g_ref[...] + be_ref[...]
        rb = lax.broadcasted_iota(jnp.int32, (BROWS, 1), 0)
        o_ref[...] = jnp.where(rb < B, xn, 0.0)

    return pl.pallas_call(
        body,
        out_shape=jax.ShapeDtypeStruct((BROWS, feat), jnp.float32),
    )(P, cinv, g, be)


def _super_mid(z2, u, deginv):
    feat = u.shape[1]

    def body(z_ref, u_ref, d_ref, o_ref):
        o_ref[...] = (z_ref[0] + z_ref[1] + u_ref[...]) * d_ref[...]

    return pl.pallas_call(
        body,
        out_shape=jax.ShapeDtypeStruct((BROWS, feat), jnp.float32),
    )(z2, u, deginv)


def _super_layer(v2, w, dinvf, W, b, g, be, res):
    cout = W.shape[1]

    def body(v_ref, w_ref, d_ref, W_ref, b_ref, g_ref, be_ref, r_ref,
             h_ref, un_ref):
        t = (v_ref[0] + v_ref[1] + w_ref[...]) * d_ref[...]
        y = jnp.maximum(
            jnp.dot(t, W_ref[...], preferred_element_type=jnp.float32)
            + b_ref[...], 0.0)
        rb = lax.broadcasted_iota(jnp.int32, (BROWS, 1), 0)
        y = jnp.where(rb < B, y, 0.0)
        m = jnp.sum(y, axis=0, keepdims=True) / B
        var = jnp.sum(y * y, axis=0, keepdims=True) / B - m * m
        xn = (y - m) * lax.rsqrt(var + EPS) * g_ref[...] + be_ref[...]
        h = jnp.where(rb < B, xn + r_ref[...], 0.0)
        h_ref[...] = h
        un_ref[...] = h * d_ref[...]

    out = jax.ShapeDtypeStruct((BROWS, cout), jnp.float32)
    return pl.pallas_call(body, out_shape=[out, out])(
        v2, w, dinvf, W, b, g, be, res)


def _head(h, Wc1, bc1, Wc2, bc2):
    def body(h_ref, w1_ref, b1_ref, w2_ref, b2_ref, o_ref):
        hid = jnp.maximum(
            jnp.dot(h_ref[...], w1_ref[...], preferred_element_type=jnp.float32)
            + b1_ref[...], 0.0)
        o_ref[...] = (jnp.dot(hid, w2_ref[...],
                              preferred_element_type=jnp.float32)
                      + b2_ref[...])

    return pl.pallas_call(
        body,
        out_shape=jax.ShapeDtypeStruct((BROWS, Wc2.shape[1]), jnp.float32),
    )(h, Wc1, bc1, Wc2, bc2)


def _pad2(a, r, c):
    return jnp.pad(a, ((0, r - a.shape[0]), (0, c - a.shape[1])))


def _pad_row(a, c):
    return jnp.pad(a, (0, c - a.shape[0])).reshape(1, c)


def _pad_idx(idx, total, fill):
    return jnp.concatenate(
        [idx.astype(jnp.int32),
         jnp.full((total - idx.shape[0],), fill, jnp.int32)])


def kernel(x, edge_index, batch, full_edge_index,
           W_gcn, b_gcn, W_gcnx, b_gcnx, W_gcny, b_gcny,
           W_g1, b_g1, W_g2, b_g2, W_g3, b_g3,
           g0, be0, g1, be1, g2, be2, g3, be3,
           Wc1, bc1, Wc2, bc2):
    src = edge_index[0].astype(jnp.int32)
    dst = edge_index[1].astype(jnp.int32)
    fs = full_edge_index[0].astype(jnp.int32)
    fd = full_edge_index[1].astype(jnp.int32)
    batch = batch.astype(jnp.int32)

    deg_idx = _pad_idx(
        jnp.concatenate([dst, NROWS + batch, NROWS + BROWS + fd]), E_DEG, N)
    cnt = _make_count_scatter(E_DEG, DEGR)(deg_idx)
    dinv, cinv, dinvf, deginvf = _prep_scalars(cnt)

    big_s = _pad_idx(src, E_BIG, 0)
    big_d = _pad_idx(dst, E_BIG, N)
    pool_s = _pad_idx(jnp.arange(N, dtype=jnp.int32), E_POOL, 0)
    pool_d = _pad_idx(batch, E_POOL, B)
    sup_s = _pad_idx(fs, E_SUP, 0)
    sup_d = _pad_idx(fd, E_SUP, B)

    prop_big = {c: _make_prop_scatter(E_BIG, NROWS, c) for c in (128, 160)}
    prop_pool = {c: _make_prop_scatter(E_POOL, BROWS, c) for c in (128, 160, 192)}
    prop_sup = {c: _make_prop_scatter(E_SUP, BROWS, c) for c in (128, 160, 192)}

    Wg = [_pad2(W_gcn, 128, 128), _pad2(W_gcnx, 128, 160), _pad2(W_gcny, 160, 192)]
    bg = [_pad_row(b_gcn, 128), _pad_row(b_gcnx, 160), _pad_row(b_gcny, 192)]
    Ws = [_pad2(W_g1, 128, 160), _pad2(W_g2, 160, 192), _pad2(W_g3, 192, 224)]
    bs = [_pad_row(b_g1, 160), _pad_row(b_g2, 192), _pad_row(b_g3, 224)]
    gs = [_pad_row(g0, 128), _pad_row(g1, 160), _pad_row(g2, 192), _pad_row(g3, 224)]
    bes = [_pad_row(be0, 128), _pad_row(be1, 160), _pad_row(be2, 192),
           _pad_row(be3, 224)]

    xp = jnp.pad(x, ((0, NROWS - N), (0, 0)))
    u = _row_scale(xp, dinv)
    xns = []
    for li in range(3):
        cin = u.shape[1]
        v2 = prop_big[cin](u, big_s, big_d)
        h, u_next = _layer_big(v2, u, dinv, Wg[li], bg[li])
        P = prop_pool[h.shape[1]](h, pool_s, pool_d)
        xns.append(_pool_finish(P, cinv, gs[li], bes[li]))
        u = u_next

    h = xns[0]
    un = _row_scale(h, dinvf)
    for li in range(3):
        cin = un.shape[1]
        z2 = prop_sup[cin](un, sup_s, sup_d)
        w = _super_mid(z2, un, deginvf)
        v2 = prop_sup[cin](w, sup_s, sup_d)
        res = (xns[li + 1] if li < 2
               else jnp.zeros((BROWS, 224), jnp.float32))
        h, un = _super_layer(v2, w, dinvf, Ws[li], bs[li],
                             gs[li + 1], bes[li + 1], res)

    logits = _head(h, _pad2(Wc1, 224, 112), _pad_row(bc1, 112),
                   _pad2(Wc2, 112, 32), _pad_row(bc2, 32))
    return logits[:B]


# speedup vs baseline: 10.0244x; 1.4781x over previous
import functools

import jax
import jax.numpy as jnp
from jax import lax
from jax.experimental import pallas as pl
from jax.experimental.pallas import tpu as pltpu
from jax.experimental.pallas import tpu_sc as plsc

N = 10000
E = 320000
B = 1000
EF = 16000

NROWS = 10240
BROWS = 1024
DEGR = NROWS + 2 * BROWS
CHUNK = 128
NW = 32
EPS = 1e-5

E_BIG = 323584
E_POOL = 12288
E_SUP = 16384
E_DEG = 348160

@functools.lru_cache(maxsize=None)
def _mesh():
    return plsc.VectorSubcoreMesh(core_axis_name="c", subcore_axis_name="s")


def _fill_rows(ref, nrows, ncols, value):
    vec = jnp.full((16,), value, dtype=ref.dtype)

    def body(i, _):
        for j in range(ncols // 16):
            ref[i, pl.ds(j * 16, 16)] = vec
        return 0

    lax.fori_loop(0, nrows, body, 0)


@functools.lru_cache(maxsize=None)
def _make_count_scatter(n_edges, acc_rows):
    cpw = n_edges // (NW * CHUNK)
    rpt = acc_rows // 16
    nzc = rpt // CHUNK

    @functools.partial(
        pl.kernel,
        mesh=_mesh(),
        compiler_params=pltpu.CompilerParams(use_tc_tiling_on_sc=False),
        out_type=jax.ShapeDtypeStruct((2, acc_rows, 16), jnp.float32),
        scratch_types=[
            pltpu.VMEM_SHARED((acc_rows, 16), jnp.float32),
            pltpu.VMEM((CHUNK, 16), jnp.float32),
            pltpu.VMEM((cpw, CHUNK), jnp.int32),
            pltpu.SemaphoreType.DMA,
        ],
    )
    def count_kernel(idx_hbm, out_hbm, acc, ones_v, idx_v, ssem):
        c = lax.axis_index("c")
        s = lax.axis_index("s")
        _fill_rows(ones_v, CHUNK, 16, 0.0)

        def zero_body(k, _):
            pltpu.sync_copy(ones_v, acc.at[pl.ds(s * rpt + k * CHUNK, CHUNK)])
            return 0

        lax.fori_loop(0, nzc, zero_body, 0)
        plsc.subcore_barrier()
        _fill_rows(ones_v, CHUNK, 16, 1.0)
        wid = s * 2 + c
        pltpu.sync_copy(idx_hbm.at[pl.ds(wid * cpw, cpw)], idx_v)

        def body(k, _):
            pltpu.async_copy(ones_v, acc.at[idx_v.at[k]], ssem, add=True)
            return 0

        lax.fori_loop(0, cpw, body, 0)

        def drain(k, _):
            pltpu.make_async_copy(ones_v, acc.at[idx_v.at[0]], ssem).wait()
            return 0

        lax.fori_loop(0, cpw, drain, 0)
        plsc.subcore_barrier()

        def wb_body(k, _):
            r0 = s * rpt + k * CHUNK
            pltpu.sync_copy(acc.at[pl.ds(r0, CHUNK)], ones_v)
            pltpu.sync_copy(ones_v, out_hbm.at[c, pl.ds(r0, CHUNK)])
            return 0

        lax.fori_loop(0, nzc, wb_body, 0)

    return count_kernel


_SPMEM_WORD_BUDGET = 2_000_000


@functools.lru_cache(maxsize=None)
def _make_prop_scatter(n_edges, acc_rows, feat, col_split=False):
    nworkers = 16 if col_split else NW
    cpw = n_edges // (nworkers * CHUNK)
    rpt = acc_rows // 16
    zc = min(CHUNK, rpt)
    nzc = rpt // zc

    def _fits(nb):
        return (acc_rows * feat
                + 16 * (nb * CHUNK * feat + 2 * cpw * CHUNK)) <= _SPMEM_WORD_BUDGET

    nbuf = max([b for b in range(1, 5) if b <= cpw and _fits(b)] or [1])
    outer = (cpw + nbuf - 1) // nbuf
    n_tables = 2 if col_split else 1

    @functools.partial(
        pl.kernel,
        mesh=_mesh(),
        compiler_params=pltpu.CompilerParams(use_tc_tiling_on_sc=False),
        out_type=jax.ShapeDtypeStruct((2, acc_rows, feat), jnp.float32),
        scratch_types=[
            pltpu.VMEM_SHARED((acc_rows, feat), jnp.float32),
            [pltpu.VMEM((CHUNK, feat), jnp.float32) for _ in range(nbuf)],
            pltpu.VMEM((cpw, CHUNK), jnp.int32),
            pltpu.VMEM((cpw, CHUNK), jnp.int32),
            [pltpu.SemaphoreType.DMA for _ in range(nbuf)],
            [pltpu.SemaphoreType.DMA for _ in range(nbuf)],
        ],
    )
    def prop_kernel(*args):
        tables = args[0:n_tables]
        sidx_hbm, didx_hbm, out_hbm, acc, rows, sidx_v, didx_v, gsem, ssem = \
            args[n_tables:]
        c = lax.axis_index("c")
        s = lax.axis_index("s")

        def start_gather(k, b):
            if col_split:
                @pl.when(c == 0)
                def _():
                    pltpu.async_copy(tables[0].at[sidx_v.at[k]], rows[b], gsem[b])

                @pl.when(c == 1)
                def _():
                    pltpu.async_copy(tables[1].at[sidx_v.at[k]], rows[b], gsem[b])
            else:
                pltpu.async_copy(tables[0].at[sidx_v.at[k]], rows[b], gsem[b])

        _fill_rows(rows[0], zc, feat, 0.0)

        def zero_body(k, _):
            pltpu.sync_copy(rows[0].at[pl.ds(0, zc)],
                            acc.at[pl.ds(s * rpt + k * zc, zc)])
            return 0

        lax.fori_loop(0, nzc, zero_body, 0)
        plsc.subcore_barrier()
        wid = s if col_split else s * 2 + c
        pltpu.sync_copy(sidx_hbm.at[pl.ds(wid * cpw, cpw)], sidx_v)
        pltpu.sync_copy(didx_hbm.at[pl.ds(wid * cpw, cpw)], didx_v)
        for b in range(nbuf):
            start_gather(b, b)

        def outer_body(o, _):
            for b in range(nbuf):
                k = o * nbuf + b

                @pl.when(k < cpw)
                def _():
                    pltpu.make_async_copy(
                        tables[0].at[sidx_v.at[k]], rows[b], gsem[b]).wait()
                    pltpu.async_copy(rows[b], acc.at[didx_v.at[k]], ssem[b],
                                     add=True)

                @pl.when(k + nbuf < cpw)
                def _():
                    pltpu.make_async_copy(
                        rows[b], acc.at[didx_v.at[k]], ssem[b]).wait()
                    start_gather(k + nbuf, b)
            return 0

        lax.fori_loop(0, outer, outer_body, 0)
        for b in range(nbuf):
            pltpu.make_async_copy(rows[b], acc.at[didx_v.at[0]], ssem[b]).wait()
        plsc.subcore_barrier()

        def wb_body(k, _):
            r0 = s * rpt + k * zc
            pltpu.sync_copy(acc.at[pl.ds(r0, zc)], rows[0].at[pl.ds(0, zc)])
            pltpu.sync_copy(rows[0].at[pl.ds(0, zc)], out_hbm.at[c, pl.ds(r0, zc)])
            return 0

        lax.fori_loop(0, nzc, wb_body, 0)

    return prop_kernel


def _prep_scalars(cnt):
    def body(cnt_ref, dinv_ref, cinv_ref, dinvf_ref, deginvf_ref):
        col = cnt_ref[0, :, 0:1] + cnt_ref[1, :, 0:1]
        rb = lax.broadcasted_iota(jnp.int32, (BROWS, 1), 0)
        rn = lax.broadcasted_iota(jnp.int32, (NROWS, 1), 0)
        deg = col[0:NROWS] + 1.0
        dinv_ref[...] = jnp.where(rn < N, lax.rsqrt(deg), 0.0)
        cb = col[NROWS:NROWS + BROWS]
        cinv_ref[...] = jnp.where(rb < B, 1.0 / jnp.maximum(cb, 1.0), 0.0)
        degf = col[NROWS + BROWS:DEGR] + 1.0
        dinvf_ref[...] = jnp.where(rb < B, lax.rsqrt(degf), 0.0)
        deginvf_ref[...] = jnp.where(rb < B, 1.0 / degf, 0.0)

    one = jax.ShapeDtypeStruct((BROWS, 1), jnp.float32)
    return pl.pallas_call(
        body,
        out_shape=[jax.ShapeDtypeStruct((NROWS, 1), jnp.float32), one, one, one],
    )(cnt)


def _row_scale(xarr, dinv):
    rows, feat = xarr.shape
    blk = 512
    grid = rows // blk

    def body(x_ref, d_ref, o_ref):
        o_ref[...] = x_ref[...] * d_ref[...]

    return pl.pallas_call(
        body,
        grid=(grid,),
        in_specs=[
            pl.BlockSpec((blk, feat), lambda i: (i, 0)),
            pl.BlockSpec((blk, 1), lambda i: (i, 0)),
        ],
        out_specs=pl.BlockSpec((blk, feat), lambda i: (i, 0)),
        out_shape=jax.ShapeDtypeStruct((rows, feat), jnp.float32),
    )(xarr, dinv)


def _row_scale_split(xarr, dinv):
    rows, feat = xarr.shape
    ch = feat // 2
    blk = 512
    grid = rows // blk

    def body(x_ref, d_ref, o_ref):
        u = x_ref[...] * d_ref[...]
        o_ref[0] = u[:, :ch]
        o_ref[1] = u[:, ch:]

    return pl.pallas_call(
        body,
        grid=(grid,),
        in_specs=[
            pl.BlockSpec((blk, feat), lambda i: (i, 0)),
            pl.BlockSpec((blk, 1), lambda i: (i, 0)),
        ],
        out_specs=pl.BlockSpec((2, blk, ch), lambda i: (0, i, 0)),
        out_shape=jax.ShapeDtypeStruct((2, rows, ch), jnp.float32),
    )(xarr, dinv)


def _layer_big(v2, u2, dinv, W, b):
    _, rows, ch_in = u2.shape
    cin = 2 * ch_in
    cout = W.shape[1]
    ch_out = cout // 2
    blk = 512
    grid = rows // blk

    def body(v_ref, u_ref, d_ref, w_ref, b_ref, h_ref, un_ref):
        p = jnp.concatenate(
            [v_ref[0] + u_ref[0], v_ref[1] + u_ref[1]], axis=1) * d_ref[...]
        h = jnp.maximum(
            jnp.dot(p, w_ref[...], preferred_element_type=jnp.float32)
            + b_ref[...], 0.0)
        h_ref[...] = h
        un = h * d_ref[...]
        un_ref[0] = un[:, :ch_out]
        un_ref[1] = un[:, ch_out:]

    return pl.pallas_call(
        body,
        grid=(grid,),
        in_specs=[
            pl.BlockSpec((2, blk, ch_in), lambda i: (0, i, 0)),
            pl.BlockSpec((2, blk, ch_in), lambda i: (0, i, 0)),
            pl.BlockSpec((blk, 1), lambda i: (i, 0)),
            pl.BlockSpec((cin, cout), lambda i: (0, 0)),
            pl.BlockSpec((1, cout), lambda i: (0, 0)),
        ],
        out_specs=[
            pl.BlockSpec((blk, cout), lambda i: (i, 0)),
            pl.BlockSpec((2, blk, ch_out), lambda i: (0, i, 0)),
        ],
        out_shape=[jax.ShapeDtypeStruct((rows, cout), jnp.float32),
                   jax.ShapeDtypeStruct((2, rows, ch_out), jnp.float32)],
    )(v2, u2, dinv, W, b)


def _pool_finish(P, cinv, g, be):
    feat = P.shape[2]

    def body(p_ref, c_ref, g_ref, be_ref, o_ref):
        pooled = (p_ref[0] + p_ref[1]) * c_ref[...]
        m = jnp.sum(pooled, axis=0, keepdims=True) / B
        var = jnp.sum(pooled * pooled, axis=0, keepdims=True) / B - m * m
        xn = (pooled - m) * lax.rsqrt(var + EPS) * g_ref[...] + be_ref[...]
        rb = lax.broadcasted_iota(jnp.int32, (BROWS, 1), 0)
        o_ref[...] = jnp.where(rb < B, xn, 0.0)

    return pl.pallas_call(
        body,
        out_shape=jax.ShapeDtypeStruct((BROWS, feat), jnp.float32),
    )(P, cinv, g, be)


def _super_mid(z2, u, deginv):
    feat = u.shape[1]

    def body(z_ref, u_ref, d_ref, o_ref):
        o_ref[...] = (z_ref[0] + z_ref[1] + u_ref[...]) * d_ref[...]

    return pl.pallas_call(
        body,
        out_shape=jax.ShapeDtypeStruct((BROWS, feat), jnp.float32),
    )(z2, u, deginv)


def _super_layer(v2, w, dinvf, W, b, g, be, res):
    cout = W.shape[1]

    def body(v_ref, w_ref, d_ref, W_ref, b_ref, g_ref, be_ref, r_ref,
             h_ref, un_ref):
        t = (v_ref[0] + v_ref[1] + w_ref[...]) * d_ref[...]
        y = jnp.maximum(
            jnp.dot(t, W_ref[...], preferred_element_type=jnp.float32)
            + b_ref[...], 0.0)
        rb = lax.broadcasted_iota(jnp.int32, (BROWS, 1), 0)
        y = jnp.where(rb < B, y, 0.0)
        m = jnp.sum(y, axis=0, keepdims=True) / B
        var = jnp.sum(y * y, axis=0, keepdims=True) / B - m * m
        xn = (y - m) * lax.rsqrt(var + EPS) * g_ref[...] + be_ref[...]
        h = jnp.where(rb < B, xn + r_ref[...], 0.0)
        h_ref[...] = h
        un_ref[...] = h * d_ref[...]

    out = jax.ShapeDtypeStruct((BROWS, cout), jnp.float32)
    return pl.pallas_call(body, out_shape=[out, out])(
        v2, w, dinvf, W, b, g, be, res)


def _head(h, Wc1, bc1, Wc2, bc2):
    def body(h_ref, w1_ref, b1_ref, w2_ref, b2_ref, o_ref):
        hid = jnp.maximum(
            jnp.dot(h_ref[...], w1_ref[...], preferred_element_type=jnp.float32)
            + b1_ref[...], 0.0)
        o_ref[...] = (jnp.dot(hid, w2_ref[...],
                              preferred_element_type=jnp.float32)
                      + b2_ref[...])

    return pl.pallas_call(
        body,
        out_shape=jax.ShapeDtypeStruct((BROWS, Wc2.shape[1]), jnp.float32),
    )(h, Wc1, bc1, Wc2, bc2)


def _pad2(a, r, c):
    return jnp.pad(a, ((0, r - a.shape[0]), (0, c - a.shape[1])))


def _pad_row(a, c):
    return jnp.pad(a, (0, c - a.shape[0])).reshape(1, c)


def _pad_idx(idx, total, fill):
    return jnp.concatenate(
        [idx.astype(jnp.int32),
         jnp.full((total - idx.shape[0],), fill, jnp.int32)]).reshape(-1, CHUNK)


def kernel(x, edge_index, batch, full_edge_index,
           W_gcn, b_gcn, W_gcnx, b_gcnx, W_gcny, b_gcny,
           W_g1, b_g1, W_g2, b_g2, W_g3, b_g3,
           g0, be0, g1, be1, g2, be2, g3, be3,
           Wc1, bc1, Wc2, bc2):
    src = edge_index[0].astype(jnp.int32)
    dst = edge_index[1].astype(jnp.int32)
    fs = full_edge_index[0].astype(jnp.int32)
    fd = full_edge_index[1].astype(jnp.int32)
    batch = batch.astype(jnp.int32)

    deg_idx = _pad_idx(
        jnp.concatenate([dst, NROWS + batch, NROWS + BROWS + fd]), E_DEG, N)
    cnt = _make_count_scatter(E_DEG, DEGR)(deg_idx)
    dinv, cinv, dinvf, deginvf = _prep_scalars(cnt)

    big_s = _pad_idx(src, E_BIG, 0)
    big_d = _pad_idx(dst, E_BIG, N)
    pool_s = _pad_idx(jnp.arange(N, dtype=jnp.int32), E_POOL, 0)
    pool_d = _pad_idx(batch, E_POOL, B)
    sup_s = _pad_idx(fs, E_SUP, 0)
    sup_d = _pad_idx(fd, E_SUP, B)

    prop_big = {c: _make_prop_scatter(E_BIG, NROWS, c // 2, col_split=True)
                for c in (128, 160)}
    prop_pool = {c: _make_prop_scatter(E_POOL, BROWS, c) for c in (128, 160, 192)}
    prop_sup = {c: _make_prop_scatter(E_SUP, BROWS, c) for c in (128, 160, 192)}

    Wg = [_pad2(W_gcn, 128, 128), _pad2(W_gcnx, 128, 160), _pad2(W_gcny, 160, 192)]
    bg = [_pad_row(b_gcn, 128), _pad_row(b_gcnx, 160), _pad_row(b_gcny, 192)]
    Ws = [_pad2(W_g1, 128, 160), _pad2(W_g2, 160, 192), _pad2(W_g3, 192, 224)]
    bs = [_pad_row(b_g1, 160), _pad_row(b_g2, 192), _pad_row(b_g3, 224)]
    gs = [_pad_row(g0, 128), _pad_row(g1, 160), _pad_row(g2, 192), _pad_row(g3, 224)]
    bes = [_pad_row(be0, 128), _pad_row(be1, 160), _pad_row(be2, 192),
           _pad_row(be3, 224)]

    xp = jnp.pad(x, ((0, NROWS - N), (0, 0)))
    u = _row_scale_split(xp, dinv)
    xns = []
    for li in range(3):
        cin = 2 * u.shape[2]
        v2 = prop_big[cin](u[0], u[1], big_s, big_d)
        h, u_next = _layer_big(v2, u, dinv, Wg[li], bg[li])
        P = prop_pool[h.shape[1]](h, pool_s, pool_d)
        xns.append(_pool_finish(P, cinv, gs[li], bes[li]))
        u = u_next

    h = xns[0]
    un = _row_scale(h, dinvf)
    for li in range(3):
        cin = un.shape[1]
        z2 = prop_sup[cin](un, sup_s, sup_d)
        w = _super_mid(z2, un, deginvf)
        v2 = prop_sup[cin](w, sup_s, sup_d)
        res = (xns[li + 1] if li < 2
               else jnp.zeros((BROWS, 224), jnp.float32))
        h, un = _super_layer(v2, w, dinvf, Ws[li], bs[li],
                             gs[li + 1], bes[li + 1], res)

    logits = _head(h, _pad2(Wc1, 224, 112), _pad_row(bc1, 112),
                   _pad2(Wc2, 112, 32), _pad_row(bc2, 32))
    return logits[:B]


# speedup vs baseline: 10.4429x; 1.0417x over previous
import functools

import jax
import jax.numpy as jnp
from jax import lax
from jax.experimental import pallas as pl
from jax.experimental.pallas import tpu as pltpu
from jax.experimental.pallas import tpu_sc as plsc

N = 10000
E = 320000
B = 1000
EF = 16000

NROWS = 10240
BROWS = 1024
DEGR = NROWS + 2 * BROWS
CHUNK = 128
NW = 32
EPS = 1e-5

E_BIG = 323584
E_POOL = 12288
E_SUP = 16384
E_DEG = 348160

@functools.lru_cache(maxsize=None)
def _mesh():
    return plsc.VectorSubcoreMesh(core_axis_name="c", subcore_axis_name="s")


def _fill_rows(ref, nrows, ncols, value):
    vec = jnp.full((16,), value, dtype=ref.dtype)

    def body(i, _):
        for j in range(ncols // 16):
            ref[i, pl.ds(j * 16, 16)] = vec
        return 0

    lax.fori_loop(0, nrows, body, 0)


@functools.lru_cache(maxsize=None)
def _make_count_scatter(n_edges, acc_rows):
    cpw = n_edges // (NW * CHUNK)
    rpt = acc_rows // 16
    nzc = rpt // CHUNK

    @functools.partial(
        pl.kernel,
        mesh=_mesh(),
        compiler_params=pltpu.CompilerParams(use_tc_tiling_on_sc=False),
        out_type=jax.ShapeDtypeStruct((2, acc_rows, 16), jnp.float32),
        scratch_types=[
            pltpu.VMEM_SHARED((acc_rows, 16), jnp.float32),
            pltpu.VMEM((CHUNK, 16), jnp.float32),
            pltpu.VMEM((cpw, CHUNK), jnp.int32),
            pltpu.SemaphoreType.DMA,
        ],
    )
    def count_kernel(idx_hbm, out_hbm, acc, ones_v, idx_v, ssem):
        c = lax.axis_index("c")
        s = lax.axis_index("s")
        _fill_rows(ones_v, CHUNK, 16, 0.0)

        def zero_body(k, _):
            pltpu.sync_copy(ones_v, acc.at[pl.ds(s * rpt + k * CHUNK, CHUNK)])
            return 0

        lax.fori_loop(0, nzc, zero_body, 0)
        plsc.subcore_barrier()
        _fill_rows(ones_v, CHUNK, 16, 1.0)
        wid = s * 2 + c
        pltpu.sync_copy(idx_hbm.at[pl.ds(wid * cpw, cpw)], idx_v)

        def body(k, _):
            pltpu.async_copy(ones_v, acc.at[idx_v.at[k]], ssem, add=True)
            return 0

        lax.fori_loop(0, cpw, body, 0)

        def drain(k, _):
            pltpu.make_async_copy(ones_v, acc.at[idx_v.at[0]], ssem).wait()
            return 0

        lax.fori_loop(0, cpw, drain, 0)
        plsc.subcore_barrier()

        def wb_body(k, _):
            r0 = s * rpt + k * CHUNK
            pltpu.sync_copy(acc.at[pl.ds(r0, CHUNK)], ones_v)
            pltpu.sync_copy(ones_v, out_hbm.at[c, pl.ds(r0, CHUNK)])
            return 0

        lax.fori_loop(0, nzc, wb_body, 0)

    return count_kernel


_SPMEM_WORD_BUDGET = 2_000_000


@functools.lru_cache(maxsize=None)
def _make_prop_scatter(n_edges, acc_rows, feat, col_split=False):
    nworkers = 16 if col_split else NW
    cpw = n_edges // (nworkers * CHUNK)
    rpt = acc_rows // 16
    zc = min(CHUNK, rpt)
    nzc = rpt // zc

    def _fits(nb):
        return (acc_rows * feat
                + 16 * (nb * CHUNK * feat + 2 * cpw * CHUNK)) <= _SPMEM_WORD_BUDGET

    nbuf = max([b for b in range(1, 5) if b <= cpw and _fits(b)] or [1])
    outer = (cpw + nbuf - 1) // nbuf
    n_tables = 2 if col_split else 1

    @functools.partial(
        pl.kernel,
        mesh=_mesh(),
        compiler_params=pltpu.CompilerParams(use_tc_tiling_on_sc=False),
        out_type=jax.ShapeDtypeStruct((2, acc_rows, feat), jnp.float32),
        scratch_types=[
            pltpu.VMEM_SHARED((acc_rows, feat), jnp.float32),
            [pltpu.VMEM((CHUNK, feat), jnp.float32) for _ in range(nbuf)],
            pltpu.VMEM((cpw, CHUNK), jnp.int32),
            pltpu.VMEM((cpw, CHUNK), jnp.int32),
            [pltpu.SemaphoreType.DMA for _ in range(nbuf)],
            [pltpu.SemaphoreType.DMA for _ in range(nbuf)],
        ],
    )
    def prop_kernel(*args):
        tables = args[0:n_tables]
        sidx_hbm, didx_hbm, out_hbm, acc, rows, sidx_v, didx_v, gsem, ssem = \
            args[n_tables:]
        c = lax.axis_index("c")
        s = lax.axis_index("s")

        def start_gather(k, b):
            if col_split:
                @pl.when(c == 0)
                def _():
                    pltpu.async_copy(tables[0].at[sidx_v.at[k]], rows[b], gsem[b])

                @pl.when(c == 1)
                def _():
                    pltpu.async_copy(tables[1].at[sidx_v.at[k]], rows[b], gsem[b])
            else:
                pltpu.async_copy(tables[0].at[sidx_v.at[k]], rows[b], gsem[b])

        _fill_rows(rows[0], zc, feat, 0.0)

        def zero_body(k, _):
            pltpu.sync_copy(rows[0].at[pl.ds(0, zc)],
                            acc.at[pl.ds(s * rpt + k * zc, zc)])
            return 0

        lax.fori_loop(0, nzc, zero_body, 0)
        plsc.subcore_barrier()
        wid = s if col_split else s * 2 + c
        pltpu.sync_copy(sidx_hbm.at[pl.ds(wid * cpw, cpw)], sidx_v)
        pltpu.sync_copy(didx_hbm.at[pl.ds(wid * cpw, cpw)], didx_v)
        for b in range(nbuf):
            start_gather(b, b)

        def outer_body(o, _):
            for b in range(nbuf):
                k = o * nbuf + b

                @pl.when(k < cpw)
                def _():
                    pltpu.make_async_copy(
                        tables[0].at[sidx_v.at[k]], rows[b], gsem[b]).wait()
                    pltpu.async_copy(rows[b], acc.at[didx_v.at[k]], ssem[b],
                                     add=True)

                @pl.when(k + nbuf < cpw)
                def _():
                    pltpu.make_async_copy(
                        rows[b], acc.at[didx_v.at[k]], ssem[b]).wait()
                    start_gather(k + nbuf, b)
            return 0

        lax.fori_loop(0, outer, outer_body, 0)
        for b in range(nbuf):
            pltpu.make_async_copy(rows[b], acc.at[didx_v.at[0]], ssem[b]).wait()
        plsc.subcore_barrier()

        def wb_body(k, _):
            r0 = s * rpt + k * zc
            pltpu.sync_copy(acc.at[pl.ds(r0, zc)], rows[0].at[pl.ds(0, zc)])
            pltpu.sync_copy(rows[0].at[pl.ds(0, zc)], out_hbm.at[c, pl.ds(r0, zc)])
            return 0

        lax.fori_loop(0, nzc, wb_body, 0)

    return prop_kernel


@functools.lru_cache(maxsize=None)
def _make_prop_pool(n_edges, pool_edges, acc_rows, pool_rows, ch):
    cpw = n_edges // (16 * CHUNK)
    ppw = pool_edges // (16 * CHUNK)
    rpt = acc_rows // 16
    prt = pool_rows // 16

    def _fits(nb):
        return ((acc_rows + pool_rows) * ch
                + 16 * (nb * CHUNK * ch + 2 * (cpw + ppw) * CHUNK)
                ) <= _SPMEM_WORD_BUDGET

    nbuf = max([b for b in range(1, 5) if _fits(b)] or [1])

    @functools.partial(
        pl.kernel,
        mesh=_mesh(),
        compiler_params=pltpu.CompilerParams(use_tc_tiling_on_sc=False),
        out_type=[jax.ShapeDtypeStruct((2, acc_rows, ch), jnp.float32),
                  jax.ShapeDtypeStruct((2, pool_rows, ch), jnp.float32)],
        scratch_types=[
            pltpu.VMEM_SHARED((acc_rows, ch), jnp.float32),
            pltpu.VMEM_SHARED((pool_rows, ch), jnp.float32),
            [pltpu.VMEM((CHUNK, ch), jnp.float32) for _ in range(nbuf)],
            pltpu.VMEM((cpw, CHUNK), jnp.int32),
            pltpu.VMEM((cpw, CHUNK), jnp.int32),
            pltpu.VMEM((ppw, CHUNK), jnp.int32),
            pltpu.VMEM((ppw, CHUNK), jnp.int32),
            [pltpu.SemaphoreType.DMA for _ in range(nbuf)],
            [pltpu.SemaphoreType.DMA for _ in range(nbuf)],
        ],
    )
    def fused_kernel(uL, uR, hL, hR, sidx_hbm, didx_hbm, psidx_hbm, pdidx_hbm,
                     out_hbm, pout_hbm, acc, pacc, rows,
                     sidx_v, didx_v, psidx_v, pdidx_v, gsem, ssem):
        c = lax.axis_index("c")
        s = lax.axis_index("s")
        _fill_rows(rows[0], CHUNK, ch, 0.0)

        def zero_body(k, _):
            pltpu.sync_copy(rows[0], acc.at[pl.ds(s * rpt + k * CHUNK, CHUNK)])
            return 0

        lax.fori_loop(0, rpt // CHUNK, zero_body, 0)
        pltpu.sync_copy(rows[0].at[pl.ds(0, prt)], pacc.at[pl.ds(s * prt, prt)])
        plsc.subcore_barrier()
        pltpu.sync_copy(sidx_hbm.at[pl.ds(s * cpw, cpw)], sidx_v)
        pltpu.sync_copy(didx_hbm.at[pl.ds(s * cpw, cpw)], didx_v)
        pltpu.sync_copy(psidx_hbm.at[pl.ds(s * ppw, ppw)], psidx_v)
        pltpu.sync_copy(pdidx_hbm.at[pl.ds(s * ppw, ppw)], pdidx_v)

        def run_phase(tl, tr, si_v, di_v, a, npw):
            def start_gather(k, b):
                @pl.when(c == 0)
                def _():
                    pltpu.async_copy(tl.at[si_v.at[k]], rows[b], gsem[b])

                @pl.when(c == 1)
                def _():
                    pltpu.async_copy(tr.at[si_v.at[k]], rows[b], gsem[b])

            for b in range(min(nbuf, npw)):
                start_gather(b, b)

            def outer_body(o, _):
                for b in range(nbuf):
                    k = o * nbuf + b

                    @pl.when(k < npw)
                    def _():
                        pltpu.make_async_copy(
                            tl.at[si_v.at[k]], rows[b], gsem[b]).wait()
                        pltpu.async_copy(rows[b], a.at[di_v.at[k]], ssem[b],
                                         add=True)

                    @pl.when(k + nbuf < npw)
                    def _():
                        pltpu.make_async_copy(
                            rows[b], a.at[di_v.at[k]], ssem[b]).wait()
                        start_gather(k + nbuf, b)
                return 0

            lax.fori_loop(0, (npw + nbuf - 1) // nbuf, outer_body, 0)
            for b in range(min(nbuf, npw)):
                pltpu.make_async_copy(rows[b], a.at[di_v.at[0]], ssem[b]).wait()

        run_phase(uL, uR, sidx_v, didx_v, acc, cpw)
        run_phase(hL, hR, psidx_v, pdidx_v, pacc, ppw)
        plsc.subcore_barrier()

        def wb_body(k, _):
            r0 = s * rpt + k * CHUNK
            pltpu.sync_copy(acc.at[pl.ds(r0, CHUNK)], rows[0])
            pltpu.sync_copy(rows[0], out_hbm.at[c, pl.ds(r0, CHUNK)])
            return 0

        lax.fori_loop(0, rpt // CHUNK, wb_body, 0)
        p0 = s * prt
        pltpu.sync_copy(pacc.at[pl.ds(p0, prt)], rows[0].at[pl.ds(0, prt)])
        pltpu.sync_copy(rows[0].at[pl.ds(0, prt)], pout_hbm.at[c, pl.ds(p0, prt)])

    return fused_kernel


def _prep_scalars(cnt):
    def body(cnt_ref, dinv_ref, cinv_ref, dinvf_ref, deginvf_ref):
        col = cnt_ref[0, :, 0:1] + cnt_ref[1, :, 0:1]
        rb = lax.broadcasted_iota(jnp.int32, (BROWS, 1), 0)
        rn = lax.broadcasted_iota(jnp.int32, (NROWS, 1), 0)
        deg = col[0:NROWS] + 1.0
        dinv_ref[...] = jnp.where(rn < N, lax.rsqrt(deg), 0.0)
        cb = col[NROWS:NROWS + BROWS]
        cinv_ref[...] = jnp.where(rb < B, 1.0 / jnp.maximum(cb, 1.0), 0.0)
        degf = col[NROWS + BROWS:DEGR] + 1.0
        dinvf_ref[...] = jnp.where(rb < B, lax.rsqrt(degf), 0.0)
        deginvf_ref[...] = jnp.where(rb < B, 1.0 / degf, 0.0)

    one = jax.ShapeDtypeStruct((BROWS, 1), jnp.float32)
    return pl.pallas_call(
        body,
        out_shape=[jax.ShapeDtypeStruct((NROWS, 1), jnp.float32), one, one, one],
    )(cnt)


def _row_scale(xarr, dinv):
    rows, feat = xarr.shape
    blk = 512
    grid = rows // blk

    def body(x_ref, d_ref, o_ref):
        o_ref[...] = x_ref[...] * d_ref[...]

    return pl.pallas_call(
        body,
        grid=(grid,),
        in_specs=[
            pl.BlockSpec((blk, feat), lambda i: (i, 0)),
            pl.BlockSpec((blk, 1), lambda i: (i, 0)),
        ],
        out_specs=pl.BlockSpec((blk, feat), lambda i: (i, 0)),
        out_shape=jax.ShapeDtypeStruct((rows, feat), jnp.float32),
    )(xarr, dinv)


def _row_scale_split(xarr, dinv):
    rows, feat = xarr.shape
    ch = feat // 2
    blk = 512
    grid = rows // blk

    def body(x_ref, d_ref, o_ref):
        u = x_ref[...] * d_ref[...]
        o_ref[0] = u[:, :ch]
        o_ref[1] = u[:, ch:]

    return pl.pallas_call(
        body,
        grid=(grid,),
        in_specs=[
            pl.BlockSpec((blk, feat), lambda i: (i, 0)),
            pl.BlockSpec((blk, 1), lambda i: (i, 0)),
        ],
        out_specs=pl.BlockSpec((2, blk, ch), lambda i: (0, i, 0)),
        out_shape=jax.ShapeDtypeStruct((2, rows, ch), jnp.float32),
    )(xarr, dinv)


def _layer_big(v2, u2, dinv, W, b):
    _, rows, ch_in = u2.shape
    cin = 2 * ch_in
    cout = W.shape[1]
    ch_out = cout // 2
    blk = 512
    grid = rows // blk

    def body(v_ref, u_ref, d_ref, w_ref, b_ref, h_ref, un_ref):
        p = jnp.concatenate(
            [v_ref[0] + u_ref[0], v_ref[1] + u_ref[1]], axis=1) * d_ref[...]
        h = jnp.maximum(
            jnp.dot(p, w_ref[...], preferred_element_type=jnp.float32)
            + b_ref[...], 0.0)
        h_ref[0] = h[:, :ch_out]
        h_ref[1] = h[:, ch_out:]
        un = h * d_ref[...]
        un_ref[0] = un[:, :ch_out]
        un_ref[1] = un[:, ch_out:]

    halves = jax.ShapeDtypeStruct((2, rows, ch_out), jnp.float32)
    return pl.pallas_call(
        body,
        grid=(grid,),
        in_specs=[
            pl.BlockSpec((2, blk, ch_in), lambda i: (0, i, 0)),
            pl.BlockSpec((2, blk, ch_in), lambda i: (0, i, 0)),
            pl.BlockSpec((blk, 1), lambda i: (i, 0)),
            pl.BlockSpec((cin, cout), lambda i: (0, 0)),
            pl.BlockSpec((1, cout), lambda i: (0, 0)),
        ],
        out_specs=[
            pl.BlockSpec((2, blk, ch_out), lambda i: (0, i, 0)),
            pl.BlockSpec((2, blk, ch_out), lambda i: (0, i, 0)),
        ],
        out_shape=[halves, halves],
    )(v2, u2, dinv, W, b)


def _pool_finish(P, cinv, g, be):
    feat = 2 * P.shape[2]

    def body(p_ref, c_ref, g_ref, be_ref, o_ref):
        pooled = jnp.concatenate([p_ref[0], p_ref[1]], axis=1) * c_ref[...]
        m = jnp.sum(pooled, axis=0, keepdims=True) / B
        var = jnp.sum(pooled * pooled, axis=0, keepdims=True) / B - m * m
        xn = (pooled - m) * lax.rsqrt(var + EPS) * g_ref[...] + be_ref[...]
        rb = lax.broadcasted_iota(jnp.int32, (BROWS, 1), 0)
        o_ref[...] = jnp.where(rb < B, xn, 0.0)

    return pl.pallas_call(
        body,
        out_shape=jax.ShapeDtypeStruct((BROWS, feat), jnp.float32),
    )(P, cinv, g, be)


def _super_mid(z2, u, deginv):
    feat = u.shape[1]

    def body(z_ref, u_ref, d_ref, o_ref):
        o_ref[...] = (z_ref[0] + z_ref[1] + u_ref[...]) * d_ref[...]

    return pl.pallas_call(
        body,
        out_shape=jax.ShapeDtypeStruct((BROWS, feat), jnp.float32),
    )(z2, u, deginv)


def _super_layer(v2, w, dinvf, W, b, g, be, res):
    cout = W.shape[1]

    def body(v_ref, w_ref, d_ref, W_ref, b_ref, g_ref, be_ref, r_ref,
             h_ref, un_ref):
        t = (v_ref[0] + v_ref[1] + w_ref[...]) * d_ref[...]
        y = jnp.maximum(
            jnp.dot(t, W_ref[...], preferred_element_type=jnp.float32)
            + b_ref[...], 0.0)
        rb = lax.broadcasted_iota(jnp.int32, (BROWS, 1), 0)
        y = jnp.where(rb < B, y, 0.0)
        m = jnp.sum(y, axis=0, keepdims=True) / B
        var = jnp.sum(y * y, axis=0, keepdims=True) / B - m * m
        xn = (y - m) * lax.rsqrt(var + EPS) * g_ref[...] + be_ref[...]
        h = jnp.where(rb < B, xn + r_ref[...], 0.0)
        h_ref[...] = h
        un_ref[...] = h * d_ref[...]

    out = jax.ShapeDtypeStruct((BROWS, cout), jnp.float32)
    return pl.pallas_call(body, out_shape=[out, out])(
        v2, w, dinvf, W, b, g, be, res)


def _head(h, Wc1, bc1, Wc2, bc2):
    def body(h_ref, w1_ref, b1_ref, w2_ref, b2_ref, o_ref):
        hid = jnp.maximum(
            jnp.dot(h_ref[...], w1_ref[...], preferred_element_type=jnp.float32)
            + b1_ref[...], 0.0)
        o_ref[...] = (jnp.dot(hid, w2_ref[...],
                              preferred_element_type=jnp.float32)
                      + b2_ref[...])

    return pl.pallas_call(
        body,
        out_shape=jax.ShapeDtypeStruct((BROWS, Wc2.shape[1]), jnp.float32),
    )(h, Wc1, bc1, Wc2, bc2)


def _pad2(a, r, c):
    return jnp.pad(a, ((0, r - a.shape[0]), (0, c - a.shape[1])))


def _pad_row(a, c):
    return jnp.pad(a, (0, c - a.shape[0])).reshape(1, c)


def _pad_idx(idx, total, fill):
    return jnp.concatenate(
        [idx.astype(jnp.int32),
         jnp.full((total - idx.shape[0],), fill, jnp.int32)]).reshape(-1, CHUNK)


def kernel(x, edge_index, batch, full_edge_index,
           W_gcn, b_gcn, W_gcnx, b_gcnx, W_gcny, b_gcny,
           W_g1, b_g1, W_g2, b_g2, W_g3, b_g3,
           g0, be0, g1, be1, g2, be2, g3, be3,
           Wc1, bc1, Wc2, bc2):
    src = edge_index[0].astype(jnp.int32)
    dst = edge_index[1].astype(jnp.int32)
    fs = full_edge_index[0].astype(jnp.int32)
    fd = full_edge_index[1].astype(jnp.int32)
    batch = batch.astype(jnp.int32)

    deg_idx = _pad_idx(
        jnp.concatenate([dst, NROWS + batch, NROWS + BROWS + fd]), E_DEG, N)
    cnt = _make_count_scatter(E_DEG, DEGR)(deg_idx)
    dinv, cinv, dinvf, deginvf = _prep_scalars(cnt)

    big_s = _pad_idx(src, E_BIG, 0)
    big_d = _pad_idx(dst, E_BIG, N)
    pool_s = _pad_idx(jnp.arange(N, dtype=jnp.int32), E_POOL, 0)
    pool_d = _pad_idx(batch, E_POOL, B)
    sup_s = _pad_idx(fs, E_SUP, 0)
    sup_d = _pad_idx(fd, E_SUP, B)

    prop_big1 = _make_prop_scatter(E_BIG, NROWS, 64, col_split=True)
    fused2 = _make_prop_pool(E_BIG, E_POOL, NROWS, BROWS, 64)
    fused3 = _make_prop_pool(E_BIG, E_POOL, NROWS, BROWS, 80)
    pool3 = _make_prop_scatter(E_POOL, BROWS, 96, col_split=True)
    prop_sup = {c: _make_prop_scatter(E_SUP, BROWS, c) for c in (128, 160, 192)}

    Wg = [_pad2(W_gcn, 128, 128), _pad2(W_gcnx, 128, 160), _pad2(W_gcny, 160, 192)]
    bg = [_pad_row(b_gcn, 128), _pad_row(b_gcnx, 160), _pad_row(b_gcny, 192)]
    Ws = [_pad2(W_g1, 128, 160), _pad2(W_g2, 160, 192), _pad2(W_g3, 192, 224)]
    bs = [_pad_row(b_g1, 160), _pad_row(b_g2, 192), _pad_row(b_g3, 224)]
    gs = [_pad_row(g0, 128), _pad_row(g1, 160), _pad_row(g2, 192), _pad_row(g3, 224)]
    bes = [_pad_row(be0, 128), _pad_row(be1, 160), _pad_row(be2, 192),
           _pad_row(be3, 224)]

    xp = jnp.pad(x, ((0, NROWS - N), (0, 0)))
    u0 = _row_scale_split(xp, dinv)
    v2 = prop_big1(u0[0], u0[1], big_s, big_d)
    h1, u1 = _layer_big(v2, u0, dinv, Wg[0], bg[0])
    v2, P1 = fused2(u1[0], u1[1], h1[0], h1[1], big_s, big_d, pool_s, pool_d)
    xn1 = _pool_finish(P1, cinv, gs[0], bes[0])
    h2, u2 = _layer_big(v2, u1, dinv, Wg[1], bg[1])
    v2, P2 = fused3(u2[0], u2[1], h2[0], h2[1], big_s, big_d, pool_s, pool_d)
    xn2 = _pool_finish(P2, cinv, gs[1], bes[1])
    h3, _ = _layer_big(v2, u2, dinv, Wg[2], bg[2])
    P3 = pool3(h3[0], h3[1], pool_s, pool_d)
    xn3 = _pool_finish(P3, cinv, gs[2], bes[2])
    xns = [xn1, xn2, xn3]

    h = xns[0]
    un = _row_scale(h, dinvf)
    for li in range(3):
        cin = un.shape[1]
        z2 = prop_sup[cin](un, sup_s, sup_d)
        w = _super_mid(z2, un, deginvf)
        v2 = prop_sup[cin](w, sup_s, sup_d)
        res = (xns[li + 1] if li < 2
               else jnp.zeros((BROWS, 224), jnp.float32))
        h, un = _super_layer(v2, w, dinvf, Ws[li], bs[li],
                             gs[li + 1], bes[li + 1], res)

    logits = _head(h, _pad2(Wc1, 224, 112), _pad_row(bc1, 112),
                   _pad2(Wc2, 112, 32), _pad_row(bc2, 32))
    return logits[:B]


# speedup vs baseline: 11.4795x; 1.0993x over previous
import functools

import jax
import jax.numpy as jnp
from jax import lax
from jax.experimental import pallas as pl
from jax.experimental.pallas import tpu as pltpu
from jax.experimental.pallas import tpu_sc as plsc

N = 10000
E = 320000
B = 1000
EF = 16000

NROWS = 10240
BROWS = 1024
DEGR = NROWS + 2 * BROWS
CHUNK = 128
NW = 32
EPS = 1e-5

E_BIG = 323584
E_POOL = 10240
E_SUP = 16384
E_DEG = 348160

@functools.lru_cache(maxsize=None)
def _mesh():
    return plsc.VectorSubcoreMesh(core_axis_name="c", subcore_axis_name="s")


def _fill_rows(ref, nrows, ncols, value):
    vec = jnp.full((16,), value, dtype=ref.dtype)

    def body(i, _):
        for j in range(ncols // 16):
            ref[i, pl.ds(j * 16, 16)] = vec
        return 0

    lax.fori_loop(0, nrows, body, 0)


@functools.lru_cache(maxsize=None)
def _make_count_scatter(n_edges, acc_rows):
    cpw = n_edges // (NW * CHUNK)
    rpt = acc_rows // 16
    nzc = rpt // CHUNK

    @functools.partial(
        pl.kernel,
        mesh=_mesh(),
        compiler_params=pltpu.CompilerParams(use_tc_tiling_on_sc=False),
        out_type=jax.ShapeDtypeStruct((2, acc_rows, 16), jnp.float32),
        scratch_types=[
            pltpu.VMEM_SHARED((acc_rows, 16), jnp.float32),
            pltpu.VMEM((CHUNK, 16), jnp.float32),
            pltpu.VMEM((cpw, CHUNK), jnp.int32),
            pltpu.SemaphoreType.DMA,
        ],
    )
    def count_kernel(idx_hbm, out_hbm, acc, ones_v, idx_v, ssem):
        c = lax.axis_index("c")
        s = lax.axis_index("s")
        _fill_rows(ones_v, CHUNK, 16, 0.0)

        def zero_body(k, _):
            pltpu.sync_copy(ones_v, acc.at[pl.ds(s * rpt + k * CHUNK, CHUNK)])
            return 0

        lax.fori_loop(0, nzc, zero_body, 0)
        plsc.subcore_barrier()
        _fill_rows(ones_v, CHUNK, 16, 1.0)
        wid = s * 2 + c
        pltpu.sync_copy(idx_hbm.at[pl.ds(wid * cpw, cpw)], idx_v)

        def body(k, _):
            pltpu.async_copy(ones_v, acc.at[idx_v.at[k]], ssem, add=True)
            return 0

        lax.fori_loop(0, cpw, body, 0)

        def drain(k, _):
            pltpu.make_async_copy(ones_v, acc.at[idx_v.at[0]], ssem).wait()
            return 0

        lax.fori_loop(0, cpw, drain, 0)
        plsc.subcore_barrier()

        def wb_body(k, _):
            r0 = s * rpt + k * CHUNK
            pltpu.sync_copy(acc.at[pl.ds(r0, CHUNK)], ones_v)
            pltpu.sync_copy(ones_v, out_hbm.at[c, pl.ds(r0, CHUNK)])
            return 0

        lax.fori_loop(0, nzc, wb_body, 0)

    return count_kernel


_SPMEM_WORD_BUDGET = 2_000_000


@functools.lru_cache(maxsize=None)
def _make_prop_scatter(n_edges, acc_rows, feat, col_split=False,
                       linear_src=False):
    nworkers = 16 if col_split else NW
    cpw = n_edges // (nworkers * CHUNK)
    rpt = acc_rows // 16
    zc = min(CHUNK, rpt)
    nzc = rpt // zc
    n_idx = 1 if linear_src else 2

    def _fits(nb):
        return (acc_rows * feat
                + 16 * (nb * CHUNK * feat + n_idx * cpw * CHUNK)
                ) <= _SPMEM_WORD_BUDGET

    nbuf = max([b for b in range(1, 5) if b <= cpw and _fits(b)] or [1])
    outer = (cpw + nbuf - 1) // nbuf
    n_tables = 2 if col_split else 1

    @functools.partial(
        pl.kernel,
        mesh=_mesh(),
        compiler_params=pltpu.CompilerParams(use_tc_tiling_on_sc=False),
        out_type=jax.ShapeDtypeStruct((2, acc_rows, feat), jnp.float32),
        scratch_types=[
            pltpu.VMEM_SHARED((acc_rows, feat), jnp.float32),
            [pltpu.VMEM((CHUNK, feat), jnp.float32) for _ in range(nbuf)],
            [pltpu.VMEM((cpw, CHUNK), jnp.int32) for _ in range(n_idx)],
            [pltpu.SemaphoreType.DMA for _ in range(nbuf)],
            [pltpu.SemaphoreType.DMA for _ in range(nbuf)],
        ],
    )
    def prop_kernel(*args):
        tables = args[0:n_tables]
        if linear_src:
            didx_hbm, out_hbm, acc, rows, (didx_v,), gsem, ssem = args[n_tables:]
        else:
            (sidx_hbm, didx_hbm, out_hbm, acc, rows, (sidx_v, didx_v),
             gsem, ssem) = args[n_tables:]
        c = lax.axis_index("c")
        s = lax.axis_index("s")
        wid = s if col_split else s * 2 + c

        def start_gather(k, b):
            if linear_src:
                row0 = (wid * cpw + k) * CHUNK

                @pl.when(c == 0)
                def _():
                    pltpu.async_copy(tables[0].at[pl.ds(row0, CHUNK)], rows[b],
                                     gsem[b])

                @pl.when(c == 1)
                def _():
                    pltpu.async_copy(tables[1].at[pl.ds(row0, CHUNK)], rows[b],
                                     gsem[b])
            elif col_split:
                @pl.when(c == 0)
                def _():
                    pltpu.async_copy(tables[0].at[sidx_v.at[k]], rows[b], gsem[b])

                @pl.when(c == 1)
                def _():
                    pltpu.async_copy(tables[1].at[sidx_v.at[k]], rows[b], gsem[b])
            else:
                pltpu.async_copy(tables[0].at[sidx_v.at[k]], rows[b], gsem[b])

        _fill_rows(rows[0], zc, feat, 0.0)

        def zero_body(k, _):
            pltpu.sync_copy(rows[0].at[pl.ds(0, zc)],
                            acc.at[pl.ds(s * rpt + k * zc, zc)])
            return 0

        lax.fori_loop(0, nzc, zero_body, 0)
        plsc.subcore_barrier()
        if not linear_src:
            pltpu.sync_copy(sidx_hbm.at[pl.ds(wid * cpw, cpw)], sidx_v)
        pltpu.sync_copy(didx_hbm.at[pl.ds(wid * cpw, cpw)], didx_v)
        for b in range(nbuf):
            start_gather(b, b)

        def outer_body(o, _):
            for b in range(nbuf):
                k = o * nbuf + b

                @pl.when(k < cpw)
                def _():
                    pltpu.make_async_copy(
                        tables[0].at[pl.ds(0, CHUNK)], rows[b], gsem[b]).wait()
                    pltpu.async_copy(rows[b], acc.at[didx_v.at[k]], ssem[b],
                                     add=True)

                @pl.when(k + nbuf < cpw)
                def _():
                    pltpu.make_async_copy(
                        rows[b], acc.at[didx_v.at[k]], ssem[b]).wait()
                    start_gather(k + nbuf, b)
            return 0

        lax.fori_loop(0, outer, outer_body, 0)
        for b in range(nbuf):
            pltpu.make_async_copy(rows[b], acc.at[didx_v.at[0]], ssem[b]).wait()
        plsc.subcore_barrier()

        def wb_body(k, _):
            r0 = s * rpt + k * zc
            pltpu.sync_copy(acc.at[pl.ds(r0, zc)], rows[0].at[pl.ds(0, zc)])
            pltpu.sync_copy(rows[0].at[pl.ds(0, zc)], out_hbm.at[c, pl.ds(r0, zc)])
            return 0

        lax.fori_loop(0, nzc, wb_body, 0)

    return prop_kernel


@functools.lru_cache(maxsize=None)
def _make_prop_pool(n_edges, pool_edges, acc_rows, pool_rows, ch):
    cpw = n_edges // (16 * CHUNK)
    ppw = pool_edges // (16 * CHUNK)
    rpt = acc_rows // 16
    prt = pool_rows // 16

    def _fits(nb):
        return ((acc_rows + pool_rows) * ch
                + 16 * (nb * CHUNK * ch + (2 * cpw + ppw) * CHUNK)
                ) <= _SPMEM_WORD_BUDGET

    nbuf = max([b for b in range(1, 5) if _fits(b)] or [1])

    @functools.partial(
        pl.kernel,
        mesh=_mesh(),
        compiler_params=pltpu.CompilerParams(use_tc_tiling_on_sc=False),
        out_type=[jax.ShapeDtypeStruct((2, acc_rows, ch), jnp.float32),
                  jax.ShapeDtypeStruct((2, pool_rows, ch), jnp.float32)],
        scratch_types=[
            pltpu.VMEM_SHARED((acc_rows, ch), jnp.float32),
            pltpu.VMEM_SHARED((pool_rows, ch), jnp.float32),
            [pltpu.VMEM((CHUNK, ch), jnp.float32) for _ in range(nbuf)],
            pltpu.VMEM((cpw, CHUNK), jnp.int32),
            pltpu.VMEM((cpw, CHUNK), jnp.int32),
            pltpu.VMEM((ppw, CHUNK), jnp.int32),
            [pltpu.SemaphoreType.DMA for _ in range(nbuf)],
            [pltpu.SemaphoreType.DMA for _ in range(nbuf)],
        ],
    )
    def fused_kernel(uL, uR, hL, hR, sidx_hbm, didx_hbm, pdidx_hbm,
                     out_hbm, pout_hbm, acc, pacc, rows,
                     sidx_v, didx_v, pdidx_v, gsem, ssem):
        c = lax.axis_index("c")
        s = lax.axis_index("s")
        _fill_rows(rows[0], CHUNK, ch, 0.0)

        def zero_body(k, _):
            pltpu.sync_copy(rows[0], acc.at[pl.ds(s * rpt + k * CHUNK, CHUNK)])
            return 0

        lax.fori_loop(0, rpt // CHUNK, zero_body, 0)
        pltpu.sync_copy(rows[0].at[pl.ds(0, prt)], pacc.at[pl.ds(s * prt, prt)])
        plsc.subcore_barrier()
        pltpu.sync_copy(sidx_hbm.at[pl.ds(s * cpw, cpw)], sidx_v)
        pltpu.sync_copy(didx_hbm.at[pl.ds(s * cpw, cpw)], didx_v)
        pltpu.sync_copy(pdidx_hbm.at[pl.ds(s * ppw, ppw)], pdidx_v)

        def run_phase(tl, tr, si_v, di_v, a, npw, base_row):
            def start_gather(k, b):
                if si_v is None:
                    row0 = (base_row + k) * CHUNK

                    @pl.when(c == 0)
                    def _():
                        pltpu.async_copy(tl.at[pl.ds(row0, CHUNK)], rows[b],
                                         gsem[b])

                    @pl.when(c == 1)
                    def _():
                        pltpu.async_copy(tr.at[pl.ds(row0, CHUNK)], rows[b],
                                         gsem[b])
                else:
                    @pl.when(c == 0)
                    def _():
                        pltpu.async_copy(tl.at[si_v.at[k]], rows[b], gsem[b])

                    @pl.when(c == 1)
                    def _():
                        pltpu.async_copy(tr.at[si_v.at[k]], rows[b], gsem[b])

            for b in range(min(nbuf, npw)):
                start_gather(b, b)

            def outer_body(o, _):
                for b in range(nbuf):
                    k = o * nbuf + b

                    @pl.when(k < npw)
                    def _():
                        pltpu.make_async_copy(
                            tl.at[pl.ds(0, CHUNK)], rows[b], gsem[b]).wait()
                        pltpu.async_copy(rows[b], a.at[di_v.at[k]], ssem[b],
                                         add=True)

                    @pl.when(k + nbuf < npw)
                    def _():
                        pltpu.make_async_copy(
                            rows[b], a.at[di_v.at[k]], ssem[b]).wait()
                        start_gather(k + nbuf, b)
                return 0

            lax.fori_loop(0, (npw + nbuf - 1) // nbuf, outer_body, 0)
            for b in range(min(nbuf, npw)):
                pltpu.make_async_copy(rows[b], a.at[di_v.at[0]], ssem[b]).wait()

        run_phase(uL, uR, sidx_v, didx_v, acc, cpw, 0)
        run_phase(hL, hR, None, pdidx_v, pacc, ppw, s * ppw)
        plsc.subcore_barrier()

        def wb_body(k, _):
            r0 = s * rpt + k * CHUNK
            pltpu.sync_copy(acc.at[pl.ds(r0, CHUNK)], rows[0])
            pltpu.sync_copy(rows[0], out_hbm.at[c, pl.ds(r0, CHUNK)])
            return 0

        lax.fori_loop(0, rpt // CHUNK, wb_body, 0)
        p0 = s * prt
        pltpu.sync_copy(pacc.at[pl.ds(p0, prt)], rows[0].at[pl.ds(0, prt)])
        pltpu.sync_copy(rows[0].at[pl.ds(0, prt)], pout_hbm.at[c, pl.ds(p0, prt)])

    return fused_kernel


def _prep_scalars(cnt):
    def body(cnt_ref, dinv_ref, cinv_ref, dinvf_ref, deginvf_ref):
        col = cnt_ref[0, :, 0:1] + cnt_ref[1, :, 0:1]
        rb = lax.broadcasted_iota(jnp.int32, (BROWS, 1), 0)
        rn = lax.broadcasted_iota(jnp.int32, (NROWS, 1), 0)
        deg = col[0:NROWS] + 1.0
        dinv_ref[...] = jnp.where(rn < N, lax.rsqrt(deg), 0.0)
        cb = col[NROWS:NROWS + BROWS]
        cinv_ref[...] = jnp.where(rb < B, 1.0 / jnp.maximum(cb, 1.0), 0.0)
        degf = col[NROWS + BROWS:DEGR] + 1.0
        dinvf_ref[...] = jnp.where(rb < B, lax.rsqrt(degf), 0.0)
        deginvf_ref[...] = jnp.where(rb < B, 1.0 / degf, 0.0)

    one = jax.ShapeDtypeStruct((BROWS, 1), jnp.float32)
    return pl.pallas_call(
        body,
        out_shape=[jax.ShapeDtypeStruct((NROWS, 1), jnp.float32), one, one, one],
    )(cnt)


def _row_scale(xarr, dinv):
    rows, feat = xarr.shape
    blk = 512
    grid = rows // blk

    def body(x_ref, d_ref, o_ref):
        o_ref[...] = x_ref[...] * d_ref[...]

    return pl.pallas_call(
        body,
        grid=(grid,),
        in_specs=[
            pl.BlockSpec((blk, feat), lambda i: (i, 0)),
            pl.BlockSpec((blk, 1), lambda i: (i, 0)),
        ],
        out_specs=pl.BlockSpec((blk, feat), lambda i: (i, 0)),
        out_shape=jax.ShapeDtypeStruct((rows, feat), jnp.float32),
    )(xarr, dinv)


def _row_scale_split(xarr, dinv):
    rows, feat = xarr.shape
    ch = feat // 2
    blk = 512
    grid = rows // blk

    def body(x_ref, d_ref, o_ref):
        u = x_ref[...] * d_ref[...]
        o_ref[0] = u[:, :ch]
        o_ref[1] = u[:, ch:]

    return pl.pallas_call(
        body,
        grid=(grid,),
        in_specs=[
            pl.BlockSpec((blk, feat), lambda i: (i, 0)),
            pl.BlockSpec((blk, 1), lambda i: (i, 0)),
        ],
        out_specs=pl.BlockSpec((2, blk, ch), lambda i: (0, i, 0)),
        out_shape=jax.ShapeDtypeStruct((2, rows, ch), jnp.float32),
    )(xarr, dinv)


def _layer_big(v2, u2, dinv, W, b):
    _, rows, ch_in = u2.shape
    cin = 2 * ch_in
    cout = W.shape[1]
    ch_out = cout // 2
    blk = 512
    grid = rows // blk

    def body(v_ref, u_ref, d_ref, w_ref, b_ref, h_ref, un_ref):
        p = jnp.concatenate(
            [v_ref[0] + u_ref[0], v_ref[1] + u_ref[1]], axis=1) * d_ref[...]
        h = jnp.maximum(
            jnp.dot(p, w_ref[...], preferred_element_type=jnp.float32)
            + b_ref[...], 0.0)
        h_ref[0] = h[:, :ch_out]
        h_ref[1] = h[:, ch_out:]
        un = h * d_ref[...]
        un_ref[0] = un[:, :ch_out]
        un_ref[1] = un[:, ch_out:]

    halves = jax.ShapeDtypeStruct((2, rows, ch_out), jnp.float32)
    return pl.pallas_call(
        body,
        grid=(grid,),
        in_specs=[
            pl.BlockSpec((2, blk, ch_in), lambda i: (0, i, 0)),
            pl.BlockSpec((2, blk, ch_in), lambda i: (0, i, 0)),
            pl.BlockSpec((blk, 1), lambda i: (i, 0)),
            pl.BlockSpec((cin, cout), lambda i: (0, 0)),
            pl.BlockSpec((1, cout), lambda i: (0, 0)),
        ],
        out_specs=[
            pl.BlockSpec((2, blk, ch_out), lambda i: (0, i, 0)),
            pl.BlockSpec((2, blk, ch_out), lambda i: (0, i, 0)),
        ],
        out_shape=[halves, halves],
    )(v2, u2, dinv, W, b)


def _pool_finish(P, cinv, g, be):
    feat = 2 * P.shape[2]

    def body(p_ref, c_ref, g_ref, be_ref, o_ref):
        pooled = jnp.concatenate([p_ref[0], p_ref[1]], axis=1) * c_ref[...]
        m = jnp.sum(pooled, axis=0, keepdims=True) / B
        var = jnp.sum(pooled * pooled, axis=0, keepdims=True) / B - m * m
        xn = (pooled - m) * lax.rsqrt(var + EPS) * g_ref[...] + be_ref[...]
        rb = lax.broadcasted_iota(jnp.int32, (BROWS, 1), 0)
        o_ref[...] = jnp.where(rb < B, xn, 0.0)

    return pl.pallas_call(
        body,
        out_shape=jax.ShapeDtypeStruct((BROWS, feat), jnp.float32),
    )(P, cinv, g, be)


def _super_mid(z2, u, deginv):
    feat = u.shape[1]

    def body(z_ref, u_ref, d_ref, o_ref):
        o_ref[...] = (z_ref[0] + z_ref[1] + u_ref[...]) * d_ref[...]

    return pl.pallas_call(
        body,
        out_shape=jax.ShapeDtypeStruct((BROWS, feat), jnp.float32),
    )(z2, u, deginv)


def _super_layer(v2, w, dinvf, W, b, g, be, res):
    cout = W.shape[1]

    def body(v_ref, w_ref, d_ref, W_ref, b_ref, g_ref, be_ref, r_ref,
             h_ref, un_ref):
        t = (v_ref[0] + v_ref[1] + w_ref[...]) * d_ref[...]
        y = jnp.maximum(
            jnp.dot(t, W_ref[...], preferred_element_type=jnp.float32)
            + b_ref[...], 0.0)
        rb = lax.broadcasted_iota(jnp.int32, (BROWS, 1), 0)
        y = jnp.where(rb < B, y, 0.0)
        m = jnp.sum(y, axis=0, keepdims=True) / B
        var = jnp.sum(y * y, axis=0, keepdims=True) / B - m * m
        xn = (y - m) * lax.rsqrt(var + EPS) * g_ref[...] + be_ref[...]
        h = jnp.where(rb < B, xn + r_ref[...], 0.0)
        h_ref[...] = h
        un_ref[...] = h * d_ref[...]

    out = jax.ShapeDtypeStruct((BROWS, cout), jnp.float32)
    return pl.pallas_call(body, out_shape=[out, out])(
        v2, w, dinvf, W, b, g, be, res)


def _head(h, Wc1, bc1, Wc2, bc2):
    def body(h_ref, w1_ref, b1_ref, w2_ref, b2_ref, o_ref):
        hid = jnp.maximum(
            jnp.dot(h_ref[...], w1_ref[...], preferred_element_type=jnp.float32)
            + b1_ref[...], 0.0)
        o_ref[...] = (jnp.dot(hid, w2_ref[...],
                              preferred_element_type=jnp.float32)
                      + b2_ref[...])

    return pl.pallas_call(
        body,
        out_shape=jax.ShapeDtypeStruct((BROWS, Wc2.shape[1]), jnp.float32),
    )(h, Wc1, bc1, Wc2, bc2)


def _pad2(a, r, c):
    return jnp.pad(a, ((0, r - a.shape[0]), (0, c - a.shape[1])))


def _pad_row(a, c):
    return jnp.pad(a, (0, c - a.shape[0])).reshape(1, c)


def _pad_idx(idx, total, fill):
    return jnp.concatenate(
        [idx.astype(jnp.int32),
         jnp.full((total - idx.shape[0],), fill, jnp.int32)]).reshape(-1, CHUNK)


def kernel(x, edge_index, batch, full_edge_index,
           W_gcn, b_gcn, W_gcnx, b_gcnx, W_gcny, b_gcny,
           W_g1, b_g1, W_g2, b_g2, W_g3, b_g3,
           g0, be0, g1, be1, g2, be2, g3, be3,
           Wc1, bc1, Wc2, bc2):
    src = edge_index[0].astype(jnp.int32)
    dst = edge_index[1].astype(jnp.int32)
    fs = full_edge_index[0].astype(jnp.int32)
    fd = full_edge_index[1].astype(jnp.int32)
    batch = batch.astype(jnp.int32)

    deg_idx = _pad_idx(
        jnp.concatenate([dst, NROWS + batch, NROWS + BROWS + fd]), E_DEG, N)
    cnt = _make_count_scatter(E_DEG, DEGR)(deg_idx)
    dinv, cinv, dinvf, deginvf = _prep_scalars(cnt)

    big_s = _pad_idx(src, E_BIG, 0)
    big_d = _pad_idx(dst, E_BIG, N)
    pool_d = _pad_idx(batch, E_POOL, B)
    sup_s = _pad_idx(fs, E_SUP, 0)
    sup_d = _pad_idx(fd, E_SUP, B)

    prop_big1 = _make_prop_scatter(E_BIG, NROWS, 64, col_split=True)
    fused2 = _make_prop_pool(E_BIG, E_POOL, NROWS, BROWS, 64)
    fused3 = _make_prop_pool(E_BIG, E_POOL, NROWS, BROWS, 80)
    pool3 = _make_prop_scatter(E_POOL, BROWS, 96, col_split=True,
                               linear_src=True)
    prop_sup = {c: _make_prop_scatter(E_SUP, BROWS, c) for c in (128, 160, 192)}

    Wg = [_pad2(W_gcn, 128, 128), _pad2(W_gcnx, 128, 160), _pad2(W_gcny, 160, 192)]
    bg = [_pad_row(b_gcn, 128), _pad_row(b_gcnx, 160), _pad_row(b_gcny, 192)]
    Ws = [_pad2(W_g1, 128, 160), _pad2(W_g2, 160, 192), _pad2(W_g3, 192, 224)]
    bs = [_pad_row(b_g1, 160), _pad_row(b_g2, 192), _pad_row(b_g3, 224)]
    gs = [_pad_row(g0, 128), _pad_row(g1, 160), _pad_row(g2, 192), _pad_row(g3, 224)]
    bes = [_pad_row(be0, 128), _pad_row(be1, 160), _pad_row(be2, 192),
           _pad_row(be3, 224)]

    xp = jnp.pad(x, ((0, NROWS - N), (0, 0)))
    u0 = _row_scale_split(xp, dinv)
    v2 = prop_big1(u0[0], u0[1], big_s, big_d)
    h1, u1 = _layer_big(v2, u0, dinv, Wg[0], bg[0])
    v2, P1 = fused2(u1[0], u1[1], h1[0], h1[1], big_s, big_d, pool_d)
    xn1 = _pool_finish(P1, cinv, gs[0], bes[0])
    h2, u2 = _layer_big(v2, u1, dinv, Wg[1], bg[1])
    v2, P2 = fused3(u2[0], u2[1], h2[0], h2[1], big_s, big_d, pool_d)
    xn2 = _pool_finish(P2, cinv, gs[1], bes[1])
    h3, _ = _layer_big(v2, u2, dinv, Wg[2], bg[2])
    P3 = pool3(h3[0], h3[1], pool_d)
    xn3 = _pool_finish(P3, cinv, gs[2], bes[2])
    xns = [xn1, xn2, xn3]

    h = xns[0]
    un = _row_scale(h, dinvf)
    for li in range(3):
        cin = un.shape[1]
        z2 = prop_sup[cin](un, sup_s, sup_d)
        w = _super_mid(z2, un, deginvf)
        v2 = prop_sup[cin](w, sup_s, sup_d)
        res = (xns[li + 1] if li < 2
               else jnp.zeros((BROWS, 224), jnp.float32))
        h, un = _super_layer(v2, w, dinvf, Ws[li], bs[li],
                             gs[li + 1], bes[li + 1], res)

    logits = _head(h, _pad2(Wc1, 224, 112), _pad_row(bc1, 112),
                   _pad2(Wc2, 112, 32), _pad_row(bc2, 32))
    return logits[:B]


# speedup vs baseline: 11.8135x; 1.0291x over previous
import functools

import jax
import jax.numpy as jnp
from jax import lax
from jax.experimental import pallas as pl
from jax.experimental.pallas import tpu as pltpu
from jax.experimental.pallas import tpu_sc as plsc

N = 10000
E = 320000
B = 1000
EF = 16000

NROWS = 10240
BROWS = 1024
DEGR = NROWS + 2 * BROWS
CHUNK = 128
NW = 32
EPS = 1e-5

E_BIG = 323584
E_POOL = 10240
E_SUP = 16384
E_DEG = 348160

@functools.lru_cache(maxsize=None)
def _mesh():
    return plsc.VectorSubcoreMesh(core_axis_name="c", subcore_axis_name="s")


def _fill_rows(ref, nrows, ncols, value):
    vec = jnp.full((16,), value, dtype=ref.dtype)

    def body(i, _):
        for j in range(ncols // 16):
            ref[i, pl.ds(j * 16, 16)] = vec
        return 0

    lax.fori_loop(0, nrows, body, 0)


@functools.lru_cache(maxsize=None)
def _make_count_scatter(n_edges, acc_rows):
    cpw = n_edges // (NW * CHUNK)
    rpt = acc_rows // 16
    nzc = rpt // CHUNK

    @functools.partial(
        pl.kernel,
        mesh=_mesh(),
        compiler_params=pltpu.CompilerParams(use_tc_tiling_on_sc=False),
        out_type=jax.ShapeDtypeStruct((2, acc_rows, 16), jnp.float32),
        scratch_types=[
            pltpu.VMEM_SHARED((acc_rows, 16), jnp.float32),
            pltpu.VMEM((CHUNK, 16), jnp.float32),
            pltpu.VMEM((cpw, CHUNK), jnp.int32),
            pltpu.SemaphoreType.DMA,
        ],
    )
    def count_kernel(idx_hbm, out_hbm, acc, ones_v, idx_v, ssem):
        c = lax.axis_index("c")
        s = lax.axis_index("s")
        _fill_rows(ones_v, CHUNK, 16, 0.0)

        def zero_body(k, _):
            pltpu.sync_copy(ones_v, acc.at[pl.ds(s * rpt + k * CHUNK, CHUNK)])
            return 0

        lax.fori_loop(0, nzc, zero_body, 0)
        plsc.subcore_barrier()
        _fill_rows(ones_v, CHUNK, 16, 1.0)
        wid = s * 2 + c
        pltpu.sync_copy(idx_hbm.at[pl.ds(wid * cpw, cpw)], idx_v)

        def body(k, _):
            pltpu.async_copy(ones_v, acc.at[idx_v.at[k]], ssem, add=True)
            return 0

        lax.fori_loop(0, cpw, body, 0)

        def drain(k, _):
            pltpu.make_async_copy(ones_v, acc.at[idx_v.at[0]], ssem).wait()
            return 0

        lax.fori_loop(0, cpw, drain, 0)
        plsc.subcore_barrier()

        def wb_body(k, _):
            r0 = s * rpt + k * CHUNK
            pltpu.sync_copy(acc.at[pl.ds(r0, CHUNK)], ones_v)
            pltpu.sync_copy(ones_v, out_hbm.at[c, pl.ds(r0, CHUNK)])
            return 0

        lax.fori_loop(0, nzc, wb_body, 0)

    return count_kernel


_SPMEM_WORD_BUDGET = 2_060_000


@functools.lru_cache(maxsize=None)
def _make_prop_scatter(n_edges, acc_rows, feat, col_split=False,
                       linear_src=False):
    nworkers = 16 if col_split else NW
    cpw = n_edges // (nworkers * CHUNK)
    rpt = acc_rows // 16
    zc = min(CHUNK, rpt)
    nzc = rpt // zc
    n_idx = 1 if linear_src else 2

    def _fits(nb):
        return (acc_rows * feat
                + 16 * (nb * CHUNK * feat + n_idx * cpw * CHUNK)
                ) <= _SPMEM_WORD_BUDGET

    nbuf = max([b for b in range(1, 5) if b <= cpw and _fits(b)] or [1])
    outer = (cpw + nbuf - 1) // nbuf
    n_tables = 2 if col_split else 1

    @functools.partial(
        pl.kernel,
        mesh=_mesh(),
        compiler_params=pltpu.CompilerParams(use_tc_tiling_on_sc=False),
        out_type=jax.ShapeDtypeStruct((2, acc_rows, feat), jnp.float32),
        scratch_types=[
            pltpu.VMEM_SHARED((acc_rows, feat), jnp.float32),
            [pltpu.VMEM((CHUNK, feat), jnp.float32) for _ in range(nbuf)],
            [pltpu.VMEM((cpw, CHUNK), jnp.int32) for _ in range(n_idx)],
            [pltpu.SemaphoreType.DMA for _ in range(nbuf)],
            [pltpu.SemaphoreType.DMA for _ in range(nbuf)],
        ],
    )
    def prop_kernel(*args):
        tables = args[0:n_tables]
        if linear_src:
            didx_hbm, out_hbm, acc, rows, (didx_v,), gsem, ssem = args[n_tables:]
        else:
            (sidx_hbm, didx_hbm, out_hbm, acc, rows, (sidx_v, didx_v),
             gsem, ssem) = args[n_tables:]
        c = lax.axis_index("c")
        s = lax.axis_index("s")
        wid = s if col_split else s * 2 + c

        def start_gather(k, b):
            if linear_src:
                row0 = (wid * cpw + k) * CHUNK

                @pl.when(c == 0)
                def _():
                    pltpu.async_copy(tables[0].at[pl.ds(row0, CHUNK)], rows[b],
                                     gsem[b])

                @pl.when(c == 1)
                def _():
                    pltpu.async_copy(tables[1].at[pl.ds(row0, CHUNK)], rows[b],
                                     gsem[b])
            elif col_split:
                @pl.when(c == 0)
                def _():
                    pltpu.async_copy(tables[0].at[sidx_v.at[k]], rows[b], gsem[b])

                @pl.when(c == 1)
                def _():
                    pltpu.async_copy(tables[1].at[sidx_v.at[k]], rows[b], gsem[b])
            else:
                pltpu.async_copy(tables[0].at[sidx_v.at[k]], rows[b], gsem[b])

        _fill_rows(rows[0], zc, feat, 0.0)

        def zero_body(k, _):
            pltpu.sync_copy(rows[0].at[pl.ds(0, zc)],
                            acc.at[pl.ds(s * rpt + k * zc, zc)])
            return 0

        lax.fori_loop(0, nzc, zero_body, 0)
        plsc.subcore_barrier()
        if not linear_src:
            pltpu.sync_copy(sidx_hbm.at[pl.ds(wid * cpw, cpw)], sidx_v)
        pltpu.sync_copy(didx_hbm.at[pl.ds(wid * cpw, cpw)], didx_v)
        for b in range(nbuf):
            start_gather(b, b)

        def outer_body(o, _):
            for b in range(nbuf):
                k = o * nbuf + b

                @pl.when(k < cpw)
                def _():
                    pltpu.make_async_copy(
                        tables[0].at[pl.ds(0, CHUNK)], rows[b], gsem[b]).wait()
                    pltpu.async_copy(rows[b], acc.at[didx_v.at[k]], ssem[b],
                                     add=True)

                @pl.when(k + nbuf < cpw)
                def _():
                    pltpu.make_async_copy(
                        rows[b], acc.at[didx_v.at[k]], ssem[b]).wait()
                    start_gather(k + nbuf, b)
            return 0

        lax.fori_loop(0, outer, outer_body, 0)
        for b in range(nbuf):
            pltpu.make_async_copy(rows[b], acc.at[didx_v.at[0]], ssem[b]).wait()
        plsc.subcore_barrier()

        def wb_body(k, _):
            r0 = s * rpt + k * zc
            pltpu.sync_copy(acc.at[pl.ds(r0, zc)], rows[0].at[pl.ds(0, zc)])
            pltpu.sync_copy(rows[0].at[pl.ds(0, zc)], out_hbm.at[c, pl.ds(r0, zc)])
            return 0

        lax.fori_loop(0, nzc, wb_body, 0)

    return prop_kernel


@functools.lru_cache(maxsize=None)
def _make_prop_pool(n_edges, pool_edges, acc_rows, pool_rows, ch):
    cpw = n_edges // (16 * CHUNK)
    ppw = pool_edges // (16 * CHUNK)
    rpt = acc_rows // 16
    prt = pool_rows // 16

    def _fits(nb):
        return ((acc_rows + pool_rows) * ch
                + 16 * (nb * CHUNK * ch + (2 * cpw + ppw) * CHUNK)
                ) <= _SPMEM_WORD_BUDGET

    nbuf = max([b for b in range(1, 5) if _fits(b)] or [1])

    @functools.partial(
        pl.kernel,
        mesh=_mesh(),
        compiler_params=pltpu.CompilerParams(use_tc_tiling_on_sc=False),
        out_type=[jax.ShapeDtypeStruct((2, acc_rows, ch), jnp.float32),
                  jax.ShapeDtypeStruct((2, pool_rows, ch), jnp.float32)],
        scratch_types=[
            pltpu.VMEM_SHARED((acc_rows, ch), jnp.float32),
            pltpu.VMEM_SHARED((pool_rows, ch), jnp.float32),
            [pltpu.VMEM((CHUNK, ch), jnp.float32) for _ in range(nbuf)],
            pltpu.VMEM((cpw, CHUNK), jnp.int32),
            pltpu.VMEM((cpw, CHUNK), jnp.int32),
            pltpu.VMEM((ppw, CHUNK), jnp.int32),
            [pltpu.SemaphoreType.DMA for _ in range(nbuf)],
            [pltpu.SemaphoreType.DMA for _ in range(nbuf)],
        ],
    )
    def fused_kernel(uL, uR, hL, hR, sidx_hbm, didx_hbm, pdidx_hbm,
                     out_hbm, pout_hbm, acc, pacc, rows,
                     sidx_v, didx_v, pdidx_v, gsem, ssem):
        c = lax.axis_index("c")
        s = lax.axis_index("s")
        _fill_rows(rows[0], CHUNK, ch, 0.0)

        def zero_body(k, _):
            pltpu.sync_copy(rows[0], acc.at[pl.ds(s * rpt + k * CHUNK, CHUNK)])
            return 0

        lax.fori_loop(0, rpt // CHUNK, zero_body, 0)
        pltpu.sync_copy(rows[0].at[pl.ds(0, prt)], pacc.at[pl.ds(s * prt, prt)])
        plsc.subcore_barrier()
        pltpu.sync_copy(sidx_hbm.at[pl.ds(s * cpw, cpw)], sidx_v)
        pltpu.sync_copy(didx_hbm.at[pl.ds(s * cpw, cpw)], didx_v)
        pltpu.sync_copy(pdidx_hbm.at[pl.ds(s * ppw, ppw)], pdidx_v)

        def run_phase(tl, tr, si_v, di_v, a, npw, base_row):
            def start_gather(k, b):
                if si_v is None:
                    row0 = (base_row + k) * CHUNK

                    @pl.when(c == 0)
                    def _():
                        pltpu.async_copy(tl.at[pl.ds(row0, CHUNK)], rows[b],
                                         gsem[b])

                    @pl.when(c == 1)
                    def _():
                        pltpu.async_copy(tr.at[pl.ds(row0, CHUNK)], rows[b],
                                         gsem[b])
                else:
                    @pl.when(c == 0)
                    def _():
                        pltpu.async_copy(tl.at[si_v.at[k]], rows[b], gsem[b])

                    @pl.when(c == 1)
                    def _():
                        pltpu.async_copy(tr.at[si_v.at[k]], rows[b], gsem[b])

            for b in range(min(nbuf, npw)):
                start_gather(b, b)

            def outer_body(o, _):
                for b in range(nbuf):
                    k = o * nbuf + b

                    @pl.when(k < npw)
                    def _():
                        pltpu.make_async_copy(
                            tl.at[pl.ds(0, CHUNK)], rows[b], gsem[b]).wait()
                        pltpu.async_copy(rows[b], a.at[di_v.at[k]], ssem[b],
                                         add=True)

                    @pl.when(k + nbuf < npw)
                    def _():
                        pltpu.make_async_copy(
                            rows[b], a.at[di_v.at[k]], ssem[b]).wait()
                        start_gather(k + nbuf, b)
                return 0

            lax.fori_loop(0, (npw + nbuf - 1) // nbuf, outer_body, 0)
            for b in range(min(nbuf, npw)):
                pltpu.make_async_copy(rows[b], a.at[di_v.at[0]], ssem[b]).wait()

        run_phase(uL, uR, sidx_v, didx_v, acc, cpw, 0)
        run_phase(hL, hR, None, pdidx_v, pacc, ppw, s * ppw)
        plsc.subcore_barrier()

        def wb_body(k, _):
            r0 = s * rpt + k * CHUNK
            pltpu.sync_copy(acc.at[pl.ds(r0, CHUNK)], rows[0])
            pltpu.sync_copy(rows[0], out_hbm.at[c, pl.ds(r0, CHUNK)])
            return 0

        lax.fori_loop(0, rpt // CHUNK, wb_body, 0)
        p0 = s * prt
        pltpu.sync_copy(pacc.at[pl.ds(p0, prt)], rows[0].at[pl.ds(0, prt)])
        pltpu.sync_copy(rows[0].at[pl.ds(0, prt)], pout_hbm.at[c, pl.ds(p0, prt)])

    return fused_kernel


def _prep_scalars(cnt):
    def body(cnt_ref, dinv_ref, cinv_ref, dinvf_ref, deginvf_ref):
        col = cnt_ref[0, :, 0:1] + cnt_ref[1, :, 0:1]
        rb = lax.broadcasted_iota(jnp.int32, (BROWS, 1), 0)
        rn = lax.broadcasted_iota(jnp.int32, (NROWS, 1), 0)
        deg = col[0:NROWS] + 1.0
        dinv_ref[...] = jnp.where(rn < N, lax.rsqrt(deg), 0.0)
        cb = col[NROWS:NROWS + BROWS]
        cinv_ref[...] = jnp.where(rb < B, 1.0 / jnp.maximum(cb, 1.0), 0.0)
        degf = col[NROWS + BROWS:DEGR] + 1.0
        dinvf_ref[...] = jnp.where(rb < B, lax.rsqrt(degf), 0.0)
        deginvf_ref[...] = jnp.where(rb < B, 1.0 / degf, 0.0)

    one = jax.ShapeDtypeStruct((BROWS, 1), jnp.float32)
    return pl.pallas_call(
        body,
        out_shape=[jax.ShapeDtypeStruct((NROWS, 1), jnp.float32), one, one, one],
    )(cnt)


def _row_scale(xarr, dinv):
    rows, feat = xarr.shape
    blk = 512
    grid = rows // blk

    def body(x_ref, d_ref, o_ref):
        o_ref[...] = x_ref[...] * d_ref[...]

    return pl.pallas_call(
        body,
        grid=(grid,),
        in_specs=[
            pl.BlockSpec((blk, feat), lambda i: (i, 0)),
            pl.BlockSpec((blk, 1), lambda i: (i, 0)),
        ],
        out_specs=pl.BlockSpec((blk, feat), lambda i: (i, 0)),
        out_shape=jax.ShapeDtypeStruct((rows, feat), jnp.float32),
    )(xarr, dinv)


def _row_scale_split(xarr, dinv):
    rows, feat = xarr.shape
    ch = feat // 2
    blk = 512
    grid = rows // blk

    def body(x_ref, d_ref, o_ref):
        u = x_ref[...] * d_ref[...]
        o_ref[0] = u[:, :ch]
        o_ref[1] = u[:, ch:]

    return pl.pallas_call(
        body,
        grid=(grid,),
        in_specs=[
            pl.BlockSpec((blk, feat), lambda i: (i, 0)),
            pl.BlockSpec((blk, 1), lambda i: (i, 0)),
        ],
        out_specs=pl.BlockSpec((2, blk, ch), lambda i: (0, i, 0)),
        out_shape=jax.ShapeDtypeStruct((2, rows, ch), jnp.float32),
    )(xarr, dinv)


def _layer_big(v2, u2, dinv, W, b):
    _, rows, ch_in = u2.shape
    cin = 2 * ch_in
    cout = W.shape[1]
    ch_out = cout // 2
    blk = 512
    grid = rows // blk

    def body(v_ref, u_ref, d_ref, w_ref, b_ref, h_ref, un_ref):
        p = jnp.concatenate(
            [v_ref[0] + u_ref[0], v_ref[1] + u_ref[1]], axis=1) * d_ref[...]
        h = jnp.maximum(
            jnp.dot(p, w_ref[...], preferred_element_type=jnp.float32)
            + b_ref[...], 0.0)
        h_ref[0] = h[:, :ch_out]
        h_ref[1] = h[:, ch_out:]
        un = h * d_ref[...]
        un_ref[0] = un[:, :ch_out]
        un_ref[1] = un[:, ch_out:]

    halves = jax.ShapeDtypeStruct((2, rows, ch_out), jnp.float32)
    return pl.pallas_call(
        body,
        grid=(grid,),
        in_specs=[
            pl.BlockSpec((2, blk, ch_in), lambda i: (0, i, 0)),
            pl.BlockSpec((2, blk, ch_in), lambda i: (0, i, 0)),
            pl.BlockSpec((blk, 1), lambda i: (i, 0)),
            pl.BlockSpec((cin, cout), lambda i: (0, 0)),
            pl.BlockSpec((1, cout), lambda i: (0, 0)),
        ],
        out_specs=[
            pl.BlockSpec((2, blk, ch_out), lambda i: (0, i, 0)),
            pl.BlockSpec((2, blk, ch_out), lambda i: (0, i, 0)),
        ],
        out_shape=[halves, halves],
    )(v2, u2, dinv, W, b)


def _pool_finish(P, cinv, g, be):
    feat = 2 * P.shape[2]

    def body(p_ref, c_ref, g_ref, be_ref, o_ref):
        pooled = jnp.concatenate([p_ref[0], p_ref[1]], axis=1) * c_ref[...]
        m = jnp.sum(pooled, axis=0, keepdims=True) / B
        var = jnp.sum(pooled * pooled, axis=0, keepdims=True) / B - m * m
        xn = (pooled - m) * lax.rsqrt(var + EPS) * g_ref[...] + be_ref[...]
        rb = lax.broadcasted_iota(jnp.int32, (BROWS, 1), 0)
        o_ref[...] = jnp.where(rb < B, xn, 0.0)

    return pl.pallas_call(
        body,
        out_shape=jax.ShapeDtypeStruct((BROWS, feat), jnp.float32),
    )(P, cinv, g, be)


def _super_mid(z2, u, deginv):
    feat = u.shape[1]

    def body(z_ref, u_ref, d_ref, o_ref):
        o_ref[...] = (z_ref[0] + z_ref[1] + u_ref[...]) * d_ref[...]

    return pl.pallas_call(
        body,
        out_shape=jax.ShapeDtypeStruct((BROWS, feat), jnp.float32),
    )(z2, u, deginv)


def _super_layer(v2, w, dinvf, W, b, g, be, res):
    cout = W.shape[1]

    def body(v_ref, w_ref, d_ref, W_ref, b_ref, g_ref, be_ref, r_ref,
             h_ref, un_ref):
        t = (v_ref[0] + v_ref[1] + w_ref[...]) * d_ref[...]
        y = jnp.maximum(
            jnp.dot(t, W_ref[...], preferred_element_type=jnp.float32)
            + b_ref[...], 0.0)
        rb = lax.broadcasted_iota(jnp.int32, (BROWS, 1), 0)
        y = jnp.where(rb < B, y, 0.0)
        m = jnp.sum(y, axis=0, keepdims=True) / B
        var = jnp.sum(y * y, axis=0, keepdims=True) / B - m * m
        xn = (y - m) * lax.rsqrt(var + EPS) * g_ref[...] + be_ref[...]
        h = jnp.where(rb < B, xn + r_ref[...], 0.0)
        h_ref[...] = h
        un_ref[...] = h * d_ref[...]

    out = jax.ShapeDtypeStruct((BROWS, cout), jnp.float32)
    return pl.pallas_call(body, out_shape=[out, out])(
        v2, w, dinvf, W, b, g, be, res)


def _head(h, Wc1, bc1, Wc2, bc2):
    def body(h_ref, w1_ref, b1_ref, w2_ref, b2_ref, o_ref):
        hid = jnp.maximum(
            jnp.dot(h_ref[...], w1_ref[...], preferred_element_type=jnp.float32)
            + b1_ref[...], 0.0)
        o_ref[...] = (jnp.dot(hid, w2_ref[...],
                              preferred_element_type=jnp.float32)
                      + b2_ref[...])

    return pl.pallas_call(
        body,
        out_shape=jax.ShapeDtypeStruct((BROWS, Wc2.shape[1]), jnp.float32),
    )(h, Wc1, bc1, Wc2, bc2)


def _pad2(a, r, c):
    return jnp.pad(a, ((0, r - a.shape[0]), (0, c - a.shape[1])))


def _pad_row(a, c):
    return jnp.pad(a, (0, c - a.shape[0])).reshape(1, c)


def _pad_idx(idx, total, fill):
    return jnp.concatenate(
        [idx.astype(jnp.int32),
         jnp.full((total - idx.shape[0],), fill, jnp.int32)]).reshape(-1, CHUNK)


def kernel(x, edge_index, batch, full_edge_index,
           W_gcn, b_gcn, W_gcnx, b_gcnx, W_gcny, b_gcny,
           W_g1, b_g1, W_g2, b_g2, W_g3, b_g3,
           g0, be0, g1, be1, g2, be2, g3, be3,
           Wc1, bc1, Wc2, bc2):
    src = edge_index[0].astype(jnp.int32)
    dst = edge_index[1].astype(jnp.int32)
    fs = full_edge_index[0].astype(jnp.int32)
    fd = full_edge_index[1].astype(jnp.int32)
    batch = batch.astype(jnp.int32)

    deg_idx = _pad_idx(
        jnp.concatenate([dst, NROWS + batch, NROWS + BROWS + fd]), E_DEG, N)
    cnt = _make_count_scatter(E_DEG, DEGR)(deg_idx)
    dinv, cinv, dinvf, deginvf = _prep_scalars(cnt)

    big_s = _pad_idx(src, E_BIG, 0)
    big_d = _pad_idx(dst, E_BIG, N)
    pool_d = _pad_idx(batch, E_POOL, B)
    sup_s = _pad_idx(fs, E_SUP, 0)
    sup_d = _pad_idx(fd, E_SUP, B)

    prop_big1 = _make_prop_scatter(E_BIG, NROWS, 64, col_split=True)
    fused2 = _make_prop_pool(E_BIG, E_POOL, NROWS, BROWS, 64)
    fused3 = _make_prop_pool(E_BIG, E_POOL, NROWS, BROWS, 80)
    pool3 = _make_prop_scatter(E_POOL, BROWS, 96, col_split=True,
                               linear_src=True)
    prop_sup = {c: _make_prop_scatter(E_SUP, BROWS, c) for c in (128, 160, 192)}

    Wg = [_pad2(W_gcn, 128, 128), _pad2(W_gcnx, 128, 160), _pad2(W_gcny, 160, 192)]
    bg = [_pad_row(b_gcn, 128), _pad_row(b_gcnx, 160), _pad_row(b_gcny, 192)]
    Ws = [_pad2(W_g1, 128, 160), _pad2(W_g2, 160, 192), _pad2(W_g3, 192, 224)]
    bs = [_pad_row(b_g1, 160), _pad_row(b_g2, 192), _pad_row(b_g3, 224)]
    gs = [_pad_row(g0, 128), _pad_row(g1, 160), _pad_row(g2, 192), _pad_row(g3, 224)]
    bes = [_pad_row(be0, 128), _pad_row(be1, 160), _pad_row(be2, 192),
           _pad_row(be3, 224)]

    xp = jnp.pad(x, ((0, NROWS - N), (0, 0)))
    u0 = _row_scale_split(xp, dinv)
    v2 = prop_big1(u0[0], u0[1], big_s, big_d)
    h1, u1 = _layer_big(v2, u0, dinv, Wg[0], bg[0])
    v2, P1 = fused2(u1[0], u1[1], h1[0], h1[1], big_s, big_d, pool_d)
    xn1 = _pool_finish(P1, cinv, gs[0], bes[0])
    h2, u2 = _layer_big(v2, u1, dinv, Wg[1], bg[1])
    v2, P2 = fused3(u2[0], u2[1], h2[0], h2[1], big_s, big_d, pool_d)
    xn2 = _pool_finish(P2, cinv, gs[1], bes[1])
    h3, _ = _layer_big(v2, u2, dinv, Wg[2], bg[2])
    P3 = pool3(h3[0], h3[1], pool_d)
    xn3 = _pool_finish(P3, cinv, gs[2], bes[2])
    xns = [xn1, xn2, xn3]

    h = xns[0]
    un = _row_scale(h, dinvf)
    for li in range(3):
        cin = un.shape[1]
        z2 = prop_sup[cin](un, sup_s, sup_d)
        w = _super_mid(z2, un, deginvf)
        v2 = prop_sup[cin](w, sup_s, sup_d)
        res = (xns[li + 1] if li < 2
               else jnp.zeros((BROWS, 224), jnp.float32))
        h, un = _super_layer(v2, w, dinvf, Ws[li], bs[li],
                             gs[li + 1], bes[li + 1], res)

    logits = _head(h, _pad2(Wc1, 224, 112), _pad_row(bc1, 112),
                   _pad2(Wc2, 112, 32), _pad_row(bc2, 32))
    return logits[:B]
